# baseline (speedup 1.0000x reference)
import functools
from typing import Any

import jax
import jax.numpy as jnp
from jax import lax
from jax.experimental import pallas as pl
from jax.experimental.pallas import tpu as pltpu
from jax.experimental.pallas import tpu_sc as plsc

N = 4096
DIM = 48
K1 = int(0.8 * N)
K2 = int(0.7 * K1)
BM = 256
GRID = N // BM


def _fused_pass(A, arrays, scales, weights, parts, outspecs, score, c_out,
                emit_bf16=False):
    n_arr = len(arrays)
    n_sc = len(scales)

    def _entry_w(e):
        return (arrays[e[0]].shape[1] if e[3] is None
                else weights[e[3][0]].shape[1])

    group_w = [_entry_w(g[0]) for g in parts]
    c_in_total = sum(group_w)
    group_off = [sum(group_w[:gi]) for gi in range(len(parts))]

    def body(*refs):
        a_ref = refs[0]
        arr_refs = refs[1:1 + n_arr]
        sc_refs = refs[1 + n_arr:1 + n_arr + n_sc]
        w_refs = refs[1 + n_arr + n_sc:1 + n_arr + n_sc + len(weights)]
        x_scr = refs[-1]
        if emit_bf16:
            out_ref, ab_ref = refs[-3], refs[-2]
        else:
            out_ref = refs[-2]
        i = pl.program_id(0)

        def load_entry(entry, row_slice=None):
            aid, coef, sid, pre = entry
            v = (arr_refs[aid][...] if row_slice is None
                 else arr_refs[aid][row_slice, :])
            if pre is not None:
                wid, bid, act, a_id = pre
                v = jnp.dot(v, w_refs[wid][...],
                            preferred_element_type=jnp.float32)
                if bid is not None:
                    v = v + w_refs[bid][...]
                if act == "prelu":
                    a = w_refs[a_id][0, 0]
                    v = jnp.where(v >= 0.0, v, a * v)
            if sid is not None:
                s = (sc_refs[sid][...] if row_slice is None
                     else sc_refs[sid][row_slice, :])
                v = v * s
            if coef != 1.0:
                v = v * coef
            return v

        @pl.when(i == 0)
        def _assemble():
            for gi, group in enumerate(parts):
                acc = None
                for entry in group:
                    v = load_entry(entry)
                    acc = v if acc is None else acc + v
                x_scr[:, group_off[gi]:group_off[gi] + group_w[gi]] = (
                    acc.astype(x_scr.dtype))

        a_blk = a_ref[...]
        if emit_bf16:
            a_blk = a_blk.astype(jnp.bfloat16)
            ab_ref[...] = a_blk
        raws = []
        for gi in range(len(parts)):
            gv = x_scr[:, group_off[gi]:group_off[gi] + group_w[gi]]
            raws.append(jnp.dot(a_blk, gv,
                                preferred_element_type=jnp.float32))

        for spec in outspecs:
            y = None
            for (gi, wid) in spec["terms"]:
                t = raws[gi] if wid is None else jnp.dot(
                    raws[gi], w_refs[wid][...],
                    preferred_element_type=jnp.float32)
                y = t if y is None else y + t
            y = y + w_refs[spec["bias_id"]][...]
            if spec["act"] == "relu":
                y = jnp.maximum(y, 0.0)
            if spec["oscale_id"] is not None:
                y = y * sc_refs[spec["oscale_id"]][pl.ds(i * BM, BM), :]
            out_ref[:, spec["col"]:spec["col"] + spec["wout"]] = y

        if score is not None:
            hn = jnp.dot(raws[score["hn_group"]], w_refs[score["wg_id"]][...],
                         preferred_element_type=jnp.float32)
            hn = hn + w_refs[score["bg_id"]][...]
            if score["hn_col"] is not None:
                out_ref[:, score["hn_col"]:score["hn_col"] + DIM] = hn
            xs = jax.nn.sigmoid(hn)
            h = None
            for entry in parts[score["h_group"]]:
                v = load_entry(entry, row_slice=pl.ds(i * BM, BM))
                h = v if h is None else h + v
            hw = jnp.dot(h, w_refs[score["wd_id"]][...],
                         preferred_element_type=jnp.float32)
            t = jnp.sum(hw * xs, axis=1, keepdims=True)
            t = t + w_refs[score["bd_id"]][...]
            out_ref[:, score["col"]:score["col"] + 1] = jax.nn.sigmoid(t)

    in_specs = [pl.BlockSpec((BM, N), lambda i: (i, 0))]
    for a in arrays:
        w = a.shape[1]
        in_specs.append(pl.BlockSpec((N, w), lambda i: (0, 0)))
    for _ in scales:
        in_specs.append(pl.BlockSpec((N, 1), lambda i: (0, 0)))
    for wgt in weights:
        in_specs.append(pl.BlockSpec(wgt.shape, lambda i: (0, 0)))

    out_specs = pl.BlockSpec((BM, c_out), lambda i: (i, 0))
    out_shape = jax.ShapeDtypeStruct((N, c_out), jnp.float32)
    if emit_bf16:
        out_specs = [out_specs, pl.BlockSpec((BM, N), lambda i: (i, 0))]
        out_shape = [out_shape, jax.ShapeDtypeStruct((N, N), jnp.bfloat16)]

    return pl.pallas_call(
        body,
        grid=(GRID,),
        in_specs=in_specs,
        out_specs=out_specs,
        out_shape=out_shape,
        scratch_shapes=[pltpu.VMEM((N, c_in_total), jnp.bfloat16)],
    )(A, *arrays, *scales, *weights)


_BR = 128


def _rank_masks_tc(scores, valid, k):
    s_col = scores.reshape(N, 1)
    s_row = scores.reshape(1, N)
    use_valid = valid is not None

    def body(*refs):
        if use_valid:
            (sc_ref, sr_ref, vr_ref, vc_ref, m_ref, sel_ref) = refs
        else:
            (sc_ref, sr_ref, m_ref, sel_ref) = refs
        si = sc_ref[...]
        sj = sr_ref[...]
        gt = (sj > si).astype(jnp.float32)
        if use_valid:
            gt = gt * vr_ref[...]
        rank = jnp.sum(gt, axis=1, keepdims=True)
        sel = rank < float(k)
        if use_valid:
            sel = sel & (vc_ref[...] > 0.0)
        m_ref[...] = jnp.where(sel, si, 0.0)
        sel_ref[...] = jnp.where(sel, 1.0, 0.0)

    in_specs = [pl.BlockSpec((_BR, 1), lambda i: (i, 0)),
                pl.BlockSpec((1, N), lambda i: (0, 0))]
    args = [s_col, s_row]
    if use_valid:
        in_specs.append(pl.BlockSpec((1, N), lambda i: (0, 0)))
        in_specs.append(pl.BlockSpec((_BR, 1), lambda i: (i, 0)))
        args.append(valid.reshape(1, N))
        args.append(valid.reshape(N, 1))

    return pl.pallas_call(
        body,
        grid=(N // _BR,),
        in_specs=in_specs,
        out_specs=[pl.BlockSpec((_BR, 1), lambda i: (i, 0)),
                   pl.BlockSpec((_BR, 1), lambda i: (i, 0))],
        out_shape=[jax.ShapeDtypeStruct((N, 1), jnp.float32),
                   jax.ShapeDtypeStruct((N, 1), jnp.float32)],
    )(*args)


_NV = N // 16


def _rank_masks_sc(scores, valid, k):
    use_valid = valid is not None
    mesh = plsc.VectorSubcoreMesh(core_axis_name="c", subcore_axis_name="s")
    n_in = 2 if use_valid else 1

    @functools.partial(
        pl.kernel, mesh=mesh,
        out_type=[jax.ShapeDtypeStruct((N,), jnp.float32),
                  jax.ShapeDtypeStruct((N,), jnp.float32)],
        scratch_types=[pltpu.VMEM((N + 32,), jnp.float32),
                       pltpu.VMEM((N,), jnp.float32),
                       pltpu.VMEM((128,), jnp.float32),
                       pltpu.VMEM((128,), jnp.float32)],
    )
    def rank_kernel(*refs):
        s_hbm = refs[0]
        v_hbm = refs[1] if use_valid else None
        m_hbm, sel_hbm = refs[n_in], refs[n_in + 1]
        sm_v, v_v, m_loc, sel_loc = refs[n_in + 2:n_in + 6]

        wid = lax.axis_index("s") * 2 + lax.axis_index("c")
        base = wid * 128
        neg = jnp.full((16,), -1.0, jnp.float32)
        pltpu.sync_copy(s_hbm, sm_v.at[pl.ds(16, N)])
        sm_v[pl.ds(0, 16)] = neg
        sm_v[pl.ds(N + 16, 16)] = neg
        if use_valid:
            pltpu.sync_copy(v_hbm, v_v)
            for q in range(_NV):
                sl = pl.ds(16 + q * 16, 16)
                sm_v[sl] = jnp.where(v_v[pl.ds(q * 16, 16)] > 0.0,
                                     sm_v[sl], -1.0)

        mines = [sm_v[pl.ds(16 + base + e * 16, 16)] for e in range(8)]

        def jbody(j, accs):
            accs = list(accs)
            jb = j * 16
            for r in range(16):
                w = sm_v[pl.ds(16 + jb + r, 16)]
                for e in range(8):
                    accs[e] = accs[e] + jnp.where(w > mines[e], 1.0, 0.0)
            return tuple(accs)

        zero = jnp.zeros((16,), jnp.float32)
        accs = list(lax.fori_loop(0, _NV, jbody, (zero,) * 8))

        for d in range(1, 16):
            w = sm_v[pl.ds(16 - d, 16)]
            for e in range(8):
                accs[e] = accs[e] + jnp.where(w > mines[e], 1.0, 0.0)

        kf = jnp.float32(k)
        for e in range(8):
            sel_e = jnp.where(accs[e] < kf, 1.0, 0.0)
            sl = pl.ds(e * 16, 16)
            sel_loc[sl] = sel_e
            m_loc[sl] = sel_e * mines[e]
        pltpu.sync_copy(m_loc, m_hbm.at[pl.ds(base, 128)])
        pltpu.sync_copy(sel_loc, sel_hbm.at[pl.ds(base, 128)])

    args = (scores, valid) if use_valid else (scores,)
    m, sel = rank_kernel(*args)
    return m.reshape(N, 1), sel.reshape(N, 1)


def kernel(A, x, params: dict[str, Any]):
    p = params

    def wt(lin):
        return lin["W"].T

    def bias(lin):
        return lin["b"].reshape(1, -1)

    x_s1, Ab = _fused_pass(
        A, [x], [],
        [wt(p["start_gcn_s1"]), bias(p["start_gcn_s1"])],
        parts=[[(0, 1.0, None, (0, None, None, None))]],
        outspecs=[dict(terms=[(0, None)], bias_id=1, act=None,
                       oscale_id=None, col=0, wout=DIM)],
        score=None, c_out=DIM, emit_bf16=True)

    is1 = p["is1"]
    w2 = [wt(p["s1_l1"]), bias(p["s1_l1"]),
          wt(is1["gcn1"]), bias(is1["gcn1"]),
          is1["disc"]["W"][0], is1["disc"]["b"].reshape(1, 1),
          wt(is1["fc"]), (is1["fc"]["b"] + is1["fc"]["bias2"]).reshape(1, -1),
          is1["fc"]["a"].reshape(1, 1)]
    pass2 = _fused_pass(
        Ab, [x_s1], [], w2,
        parts=[[(0, 1.0, None, (6, 7, "prelu", 8))], [(0, 1.0, None, None)]],
        outspecs=[dict(terms=[(1, 0)], bias_id=1, act="relu",
                       oscale_id=None, col=0, wout=DIM)],
        score=dict(hn_group=0, wg_id=2, bg_id=3, wd_id=4, bd_id=5,
                   h_group=0, col=DIM, hn_col=None),
        c_out=DIM + 1)
    x_s1a = pass2[:, 0:DIM]
    scores1 = pass2[:, DIM]
    m1, sel1 = _rank_masks_sc(scores1, None, K1)

    x_s2 = _fused_pass(
        Ab, [x_s1], [m1], [wt(p["start_gcn_s2"]), bias(p["start_gcn_s2"])],
        parts=[[(0, 1.0, 0, None)]],
        outspecs=[dict(terms=[(0, 0)], bias_id=1, act=None,
                       oscale_id=None, col=0, wout=DIM)],
        score=None, c_out=DIM)

    is2 = p["is2"]
    w4 = [wt(p["s2_l1"]), bias(p["s2_l1"]),
          wt(is2["gcn1"]), bias(is2["gcn1"]),
          is2["disc"]["W"][0], is2["disc"]["b"].reshape(1, 1),
          wt(is2["fc"]), (is2["fc"]["b"] + is2["fc"]["bias2"]).reshape(1, -1),
          is2["fc"]["a"].reshape(1, 1)]
    pass4 = _fused_pass(
        Ab, [x_s2], [sel1], w4,
        parts=[[(0, 1.0, 0, (6, 7, "prelu", 8))], [(0, 1.0, 0, None)]],
        outspecs=[dict(terms=[(1, 0)], bias_id=1, act="relu",
                       oscale_id=None, col=DIM, wout=DIM)],
        score=dict(hn_group=0, wg_id=2, bg_id=3, wd_id=4, bd_id=5,
                   h_group=0, col=2 * DIM, hn_col=0),
        c_out=2 * DIM + 1)
    xdown2 = pass4[:, 0:DIM]
    x_s2a = pass4[:, DIM:2 * DIM]
    scores2 = pass4[:, 2 * DIM]
    m2, sel2 = _rank_masks_sc(scores2, sel1.reshape(N), K2)

    x_s3a = _fused_pass(
        Ab, [x_s2], [m2], [wt(p["s3_l1"]), bias(p["s3_l1"])],
        parts=[[(0, 1.0, 0, None)]],
        outspecs=[dict(terms=[(0, 0)], bias_id=1, act="relu",
                       oscale_id=None, col=0, wout=DIM)],
        score=None, c_out=DIM)

    def cross_pass(xs1, xs2, xs3, wp12, wu21, wp23, wu32):
        w = [wt(wp12), bias(wp12), wt(wu21), bias(wu21),
             wt(wp23), bias(wp23), wt(wu32), bias(wu32)]
        out = _fused_pass(
            Ab, [xs1, xs2, xs3], [sel1, sel2, m1, m2], w,
            parts=[[(0, 1.0, None, None)], [(1, 1.0, 0, None)],
                   [(2, 1.0, 1, None)]],
            outspecs=[
                dict(terms=[(0, 0)], bias_id=1, act=None, oscale_id=2,
                     col=0, wout=DIM),
                dict(terms=[(1, 2)], bias_id=3, act=None, oscale_id=None,
                     col=DIM, wout=DIM),
                dict(terms=[(1, 4)], bias_id=5, act=None, oscale_id=3,
                     col=2 * DIM, wout=DIM),
                dict(terms=[(2, 6)], bias_id=7, act=None, oscale_id=None,
                     col=3 * DIM, wout=DIM),
            ],
            score=None, c_out=4 * DIM)
        return (out[:, 0:DIM], out[:, DIM:2 * DIM],
                out[:, 2 * DIM:3 * DIM], out[:, 3 * DIM:4 * DIM])

    x12, x21, x23, x32 = cross_pass(
        x_s1a, x_s2a, x_s3a,
        p["pool_s12_1"], p["unpool_s21_1"], p["pool_s23_1"], p["unpool_s32_1"])

    def tri_pass(arrays, scales, groups, l1, l2, l3):
        w = [wt(l1), bias(l1), wt(l2), bias(l2), wt(l3), bias(l3)]
        out = _fused_pass(
            Ab, arrays, scales, w, parts=groups,
            outspecs=[
                dict(terms=[(0, 0)], bias_id=1, act="relu", oscale_id=None,
                     col=0, wout=DIM),
                dict(terms=[(1, 2)], bias_id=3, act="relu", oscale_id=None,
                     col=DIM, wout=DIM),
                dict(terms=[(2, 4)], bias_id=5, act="relu", oscale_id=None,
                     col=2 * DIM, wout=DIM),
            ],
            score=None, c_out=3 * DIM)
        return out[:, 0:DIM], out[:, DIM:2 * DIM], out[:, 2 * DIM:3 * DIM]

    x_s1b, x_s2b, x_s3b = tri_pass(
        [x_s1a, x21, x_s1, x_s2a, x12, x32, x_s2, x_s3a, x23],
        [sel1, sel2, m2],
        [
            [(0, 1.0, None, None), (1, 1.0, None, None), (2, 1.0, None, None)],
            [(3, 1.0, 0, None), (4, 0.5, None, None), (5, 0.5, 0, None),
             (6, 1.0, 0, None)],
            [(7, 1.0, 1, None), (8, 1.0, None, None), (6, 1.0, 2, None)],
        ],
        p["s1_l2"], p["s2_l2"], p["s3_l2"])

    x12b, x21b, x23b, x32b = cross_pass(
        x_s1b, x_s2b, x_s3b,
        p["pool_s12_2"], p["unpool_s21_2"], p["pool_s23_2"], p["unpool_s32_2"])

    x_s1f, x_s2f, x_s3f = tri_pass(
        [x_s1b, x21b, x_s2b, x12b, x32b, x_s3b, x23b],
        [sel1, sel2],
        [
            [(0, 1.0, None, None), (1, 0.05, None, None)],
            [(2, 1.0, 0, None), (3, 0.025, None, None), (4, 0.025, 0, None)],
            [(5, 1.0, 1, None), (6, 0.05, None, None)],
        ],
        p["s1_l3"], p["s2_l3"], p["s3_l3"])

    u32e = _fused_pass(
        Ab, [x_s3f], [sel2],
        [wt(p["unpool_s32_end"]), bias(p["unpool_s32_end"])],
        parts=[[(0, 1.0, 0, None)]],
        outspecs=[dict(terms=[(0, 0)], bias_id=1, act=None,
                       oscale_id=None, col=0, wout=DIM)],
        score=None, c_out=DIM)

    x_s2out = _fused_pass(
        Ab, [x_s2f, u32e, xdown2], [sel1],
        [wt(p["unpool_s21_end"]), bias(p["unpool_s21_end"])],
        parts=[[(0, 1.0, 0, None), (1, 1.0, 0, None), (2, 1.0, 0, None)]],
        outspecs=[dict(terms=[(0, 0)], bias_id=1, act=None,
                       oscale_id=None, col=0, wout=DIM)],
        score=None, c_out=DIM)

    wend = wt(p["end_gcn"])
    out = _fused_pass(
        Ab, [x_s1f, x_s2out], [],
        [wend[0:DIM, :], wend[DIM:2 * DIM, :], bias(p["end_gcn"])],
        parts=[[(0, 1.0, None, None)], [(1, 1.0, None, None)]],
        outspecs=[dict(terms=[(0, 0), (1, 1)], bias_id=2, act=None,
                       oscale_id=None, col=0, wout=256)],
        score=None, c_out=256)
    return out


# speedup vs baseline: 1.2263x; 1.2263x over previous
import functools
from typing import Any

import jax
import jax.numpy as jnp
from jax import lax
from jax.experimental import pallas as pl
from jax.experimental.pallas import tpu as pltpu
from jax.experimental.pallas import tpu_sc as plsc

N = 4096
DIM = 48
K1 = int(0.8 * N)
K2 = int(0.7 * K1)
BM = 256
GRID = N // BM


def _fused_pass(A, arrays, scales, weights, parts, outspecs, score, c_out,
                emit_bf16=False):
    n_arr = len(arrays)
    n_sc = len(scales)

    def _entry_w(e):
        return (arrays[e[0]].shape[1] if e[3] is None
                else weights[e[3][0]].shape[1])

    group_w = [_entry_w(g[0]) for g in parts]
    c_in_total = sum(group_w)
    group_off = [sum(group_w[:gi]) for gi in range(len(parts))]

    def body(*refs):
        a_ref = refs[0]
        arr_refs = refs[1:1 + n_arr]
        sc_refs = refs[1 + n_arr:1 + n_arr + n_sc]
        w_refs = refs[1 + n_arr + n_sc:1 + n_arr + n_sc + len(weights)]
        x_scr = refs[-1]
        if emit_bf16:
            out_ref, ab_ref = refs[-3], refs[-2]
        else:
            out_ref = refs[-2]
        i = pl.program_id(0)

        def load_entry(entry, row_slice=None):
            aid, coef, sid, pre = entry
            v = (arr_refs[aid][...] if row_slice is None
                 else arr_refs[aid][row_slice, :])
            if pre is not None:
                wid, bid, act, a_id = pre
                v = jnp.dot(v, w_refs[wid][...],
                            preferred_element_type=jnp.float32)
                if bid is not None:
                    v = v + w_refs[bid][...]
                if act == "prelu":
                    a = w_refs[a_id][0, 0]
                    v = jnp.where(v >= 0.0, v, a * v)
            if sid is not None:
                s = (sc_refs[sid][...] if row_slice is None
                     else sc_refs[sid][row_slice, :])
                v = v * s
            if coef != 1.0:
                v = v * coef
            return v

        @pl.when(i == 0)
        def _assemble():
            for gi, group in enumerate(parts):
                acc = None
                for entry in group:
                    v = load_entry(entry)
                    acc = v if acc is None else acc + v
                x_scr[:, group_off[gi]:group_off[gi] + group_w[gi]] = (
                    acc.astype(x_scr.dtype))

        a_blk = a_ref[...]
        if emit_bf16:
            a_blk = a_blk.astype(jnp.bfloat16)
            ab_ref[...] = a_blk
        raws = []
        for gi in range(len(parts)):
            gv = x_scr[:, group_off[gi]:group_off[gi] + group_w[gi]]
            raws.append(jnp.dot(a_blk, gv,
                                preferred_element_type=jnp.float32))

        for spec in outspecs:
            y = None
            for (gi, wid) in spec["terms"]:
                t = raws[gi] if wid is None else jnp.dot(
                    raws[gi], w_refs[wid][...],
                    preferred_element_type=jnp.float32)
                y = t if y is None else y + t
            y = y + w_refs[spec["bias_id"]][...]
            if spec["act"] == "relu":
                y = jnp.maximum(y, 0.0)
            if spec["oscale_id"] is not None:
                y = y * sc_refs[spec["oscale_id"]][pl.ds(i * BM, BM), :]
            out_ref[:, spec["col"]:spec["col"] + spec["wout"]] = y

        if score is not None:
            hn = jnp.dot(raws[score["hn_group"]], w_refs[score["wg_id"]][...],
                         preferred_element_type=jnp.float32)
            hn = hn + w_refs[score["bg_id"]][...]
            if score["hn_col"] is not None:
                out_ref[:, score["hn_col"]:score["hn_col"] + DIM] = hn
            xs = jax.nn.sigmoid(hn)
            h = None
            for entry in parts[score["h_group"]]:
                v = load_entry(entry, row_slice=pl.ds(i * BM, BM))
                h = v if h is None else h + v
            hw = jnp.dot(h, w_refs[score["wd_id"]][...],
                         preferred_element_type=jnp.float32)
            t = jnp.sum(hw * xs, axis=1, keepdims=True)
            t = t + w_refs[score["bd_id"]][...]
            out_ref[:, score["col"]:score["col"] + 1] = jax.nn.sigmoid(t)

    in_specs = [pl.BlockSpec((BM, N), lambda i: (i, 0))]
    for a in arrays:
        w = a.shape[1]
        in_specs.append(pl.BlockSpec((N, w), lambda i: (0, 0)))
    for _ in scales:
        in_specs.append(pl.BlockSpec((N, 1), lambda i: (0, 0)))
    for wgt in weights:
        in_specs.append(pl.BlockSpec(wgt.shape, lambda i: (0, 0)))

    out_specs = pl.BlockSpec((BM, c_out), lambda i: (i, 0))
    out_shape = jax.ShapeDtypeStruct((N, c_out), jnp.float32)
    if emit_bf16:
        out_specs = [out_specs, pl.BlockSpec((BM, N), lambda i: (i, 0))]
        out_shape = [out_shape, jax.ShapeDtypeStruct((N, N), jnp.bfloat16)]

    return pl.pallas_call(
        body,
        grid=(GRID,),
        in_specs=in_specs,
        out_specs=out_specs,
        out_shape=out_shape,
        scratch_shapes=[pltpu.VMEM((N, c_in_total), jnp.bfloat16)],
    )(A, *arrays, *scales, *weights)


_BR = 128


def _rank_masks_tc(scores, valid, k):
    s_col = scores.reshape(N, 1)
    s_row = scores.reshape(1, N)
    use_valid = valid is not None

    def body(*refs):
        if use_valid:
            (sc_ref, sr_ref, vr_ref, vc_ref, m_ref, sel_ref) = refs
        else:
            (sc_ref, sr_ref, m_ref, sel_ref) = refs
        si = sc_ref[...]
        sj = sr_ref[...]
        gt = (sj > si).astype(jnp.float32)
        if use_valid:
            gt = gt * vr_ref[...]
        rank = jnp.sum(gt, axis=1, keepdims=True)
        sel = rank < float(k)
        if use_valid:
            sel = sel & (vc_ref[...] > 0.0)
        m_ref[...] = jnp.where(sel, si, 0.0)
        sel_ref[...] = jnp.where(sel, 1.0, 0.0)

    in_specs = [pl.BlockSpec((_BR, 1), lambda i: (i, 0)),
                pl.BlockSpec((1, N), lambda i: (0, 0))]
    args = [s_col, s_row]
    if use_valid:
        in_specs.append(pl.BlockSpec((1, N), lambda i: (0, 0)))
        in_specs.append(pl.BlockSpec((_BR, 1), lambda i: (i, 0)))
        args.append(valid.reshape(1, N))
        args.append(valid.reshape(N, 1))

    return pl.pallas_call(
        body,
        grid=(N // _BR,),
        in_specs=in_specs,
        out_specs=[pl.BlockSpec((_BR, 1), lambda i: (i, 0)),
                   pl.BlockSpec((_BR, 1), lambda i: (i, 0))],
        out_shape=[jax.ShapeDtypeStruct((N, 1), jnp.float32),
                   jax.ShapeDtypeStruct((N, 1), jnp.float32)],
    )(*args)


_NV = N // 16


def _rank_masks_sc(scores, valid, k):
    use_valid = valid is not None
    mesh = plsc.VectorSubcoreMesh(core_axis_name="c", subcore_axis_name="s")
    n_in = 2 if use_valid else 1

    @functools.partial(
        pl.kernel, mesh=mesh,
        out_type=[jax.ShapeDtypeStruct((N,), jnp.float32),
                  jax.ShapeDtypeStruct((N,), jnp.float32)],
        scratch_types=[pltpu.VMEM((N + 32,), jnp.float32),
                       pltpu.VMEM((N,), jnp.float32),
                       pltpu.VMEM((128,), jnp.float32),
                       pltpu.VMEM((128,), jnp.float32)],
    )
    def rank_kernel(*refs):
        s_hbm = refs[0]
        v_hbm = refs[1] if use_valid else None
        m_hbm, sel_hbm = refs[n_in], refs[n_in + 1]
        sm_v, v_v, m_loc, sel_loc = refs[n_in + 2:n_in + 6]

        wid = lax.axis_index("s") * 2 + lax.axis_index("c")
        base = wid * 128
        neg = jnp.full((16,), -1.0, jnp.float32)
        pltpu.sync_copy(s_hbm, sm_v.at[pl.ds(16, N)])
        sm_v[pl.ds(0, 16)] = neg
        sm_v[pl.ds(N + 16, 16)] = neg
        if use_valid:
            pltpu.sync_copy(v_hbm, v_v)
            for q in range(_NV):
                sl = pl.ds(16 + q * 16, 16)
                sm_v[sl] = jnp.where(v_v[pl.ds(q * 16, 16)] > 0.0,
                                     sm_v[sl], -1.0)

        mines = [sm_v[pl.ds(16 + base + e * 16, 16)] for e in range(8)]

        def jbody(j, carry):
            acc_a, acc_b = list(carry[0]), list(carry[1])
            jb = j * 32
            for r in range(0, 32, 2):
                wa = sm_v[pl.ds(16 + jb + r, 16)]
                wb = sm_v[pl.ds(16 + jb + r + 1, 16)]
                for e in range(8):
                    acc_a[e] = jnp.where(wa > mines[e],
                                         acc_a[e] + 1.0, acc_a[e])
                    acc_b[e] = jnp.where(wb > mines[e],
                                         acc_b[e] + 1.0, acc_b[e])
            return (tuple(acc_a), tuple(acc_b))

        zero = jnp.zeros((16,), jnp.float32)
        acc_a, acc_b = lax.fori_loop(0, _NV // 2, jbody,
                                     ((zero,) * 8, (zero,) * 8))
        accs = [a + b for a, b in zip(acc_a, acc_b)]

        for d in range(1, 16):
            w = sm_v[pl.ds(16 - d, 16)]
            for e in range(8):
                accs[e] = accs[e] + jnp.where(w > mines[e], 1.0, 0.0)

        kf = jnp.float32(k)
        for e in range(8):
            sel_e = jnp.where(accs[e] < kf, 1.0, 0.0)
            sl = pl.ds(e * 16, 16)
            sel_loc[sl] = sel_e
            m_loc[sl] = sel_e * mines[e]
        pltpu.sync_copy(m_loc, m_hbm.at[pl.ds(base, 128)])
        pltpu.sync_copy(sel_loc, sel_hbm.at[pl.ds(base, 128)])

    args = (scores, valid) if use_valid else (scores,)
    m, sel = rank_kernel(*args)
    return m.reshape(N, 1), sel.reshape(N, 1)


def kernel(A, x, params: dict[str, Any]):
    p = params

    def wt(lin):
        return lin["W"].T

    def bias(lin):
        return lin["b"].reshape(1, -1)

    x_s1, Ab = _fused_pass(
        A, [x], [],
        [wt(p["start_gcn_s1"]), bias(p["start_gcn_s1"])],
        parts=[[(0, 1.0, None, (0, None, None, None))]],
        outspecs=[dict(terms=[(0, None)], bias_id=1, act=None,
                       oscale_id=None, col=0, wout=DIM)],
        score=None, c_out=DIM, emit_bf16=True)

    is1 = p["is1"]
    w2 = [wt(p["s1_l1"]), bias(p["s1_l1"]),
          wt(is1["gcn1"]), bias(is1["gcn1"]),
          is1["disc"]["W"][0], is1["disc"]["b"].reshape(1, 1),
          wt(is1["fc"]), (is1["fc"]["b"] + is1["fc"]["bias2"]).reshape(1, -1),
          is1["fc"]["a"].reshape(1, 1)]
    pass2 = _fused_pass(
        Ab, [x_s1], [], w2,
        parts=[[(0, 1.0, None, (6, 7, "prelu", 8))], [(0, 1.0, None, None)]],
        outspecs=[dict(terms=[(1, 0)], bias_id=1, act="relu",
                       oscale_id=None, col=0, wout=DIM)],
        score=dict(hn_group=0, wg_id=2, bg_id=3, wd_id=4, bd_id=5,
                   h_group=0, col=DIM, hn_col=None),
        c_out=DIM + 1)
    x_s1a = pass2[:, 0:DIM]
    scores1 = pass2[:, DIM]
    m1, sel1 = _rank_masks_sc(scores1, None, K1)

    x_s2 = _fused_pass(
        Ab, [x_s1], [m1], [wt(p["start_gcn_s2"]), bias(p["start_gcn_s2"])],
        parts=[[(0, 1.0, 0, None)]],
        outspecs=[dict(terms=[(0, 0)], bias_id=1, act=None,
                       oscale_id=None, col=0, wout=DIM)],
        score=None, c_out=DIM)

    is2 = p["is2"]
    w4 = [wt(p["s2_l1"]), bias(p["s2_l1"]),
          wt(is2["gcn1"]), bias(is2["gcn1"]),
          is2["disc"]["W"][0], is2["disc"]["b"].reshape(1, 1),
          wt(is2["fc"]), (is2["fc"]["b"] + is2["fc"]["bias2"]).reshape(1, -1),
          is2["fc"]["a"].reshape(1, 1)]
    pass4 = _fused_pass(
        Ab, [x_s2], [sel1], w4,
        parts=[[(0, 1.0, 0, (6, 7, "prelu", 8))], [(0, 1.0, 0, None)]],
        outspecs=[dict(terms=[(1, 0)], bias_id=1, act="relu",
                       oscale_id=None, col=DIM, wout=DIM)],
        score=dict(hn_group=0, wg_id=2, bg_id=3, wd_id=4, bd_id=5,
                   h_group=0, col=2 * DIM, hn_col=0),
        c_out=2 * DIM + 1)
    xdown2 = pass4[:, 0:DIM]
    x_s2a = pass4[:, DIM:2 * DIM]
    scores2 = pass4[:, 2 * DIM]
    m2, sel2 = _rank_masks_sc(scores2, sel1.reshape(N), K2)

    x_s3a = _fused_pass(
        Ab, [x_s2], [m2], [wt(p["s3_l1"]), bias(p["s3_l1"])],
        parts=[[(0, 1.0, 0, None)]],
        outspecs=[dict(terms=[(0, 0)], bias_id=1, act="relu",
                       oscale_id=None, col=0, wout=DIM)],
        score=None, c_out=DIM)

    def cross_pass(xs1, xs2, xs3, wp12, wu21, wp23, wu32):
        w = [wt(wp12), bias(wp12), wt(wu21), bias(wu21),
             wt(wp23), bias(wp23), wt(wu32), bias(wu32)]
        out = _fused_pass(
            Ab, [xs1, xs2, xs3], [sel1, sel2, m1, m2], w,
            parts=[[(0, 1.0, None, None)], [(1, 1.0, 0, None)],
                   [(2, 1.0, 1, None)]],
            outspecs=[
                dict(terms=[(0, 0)], bias_id=1, act=None, oscale_id=2,
                     col=0, wout=DIM),
                dict(terms=[(1, 2)], bias_id=3, act=None, oscale_id=None,
                     col=DIM, wout=DIM),
                dict(terms=[(1, 4)], bias_id=5, act=None, oscale_id=3,
                     col=2 * DIM, wout=DIM),
                dict(terms=[(2, 6)], bias_id=7, act=None, oscale_id=None,
                     col=3 * DIM, wout=DIM),
            ],
            score=None, c_out=4 * DIM)
        return (out[:, 0:DIM], out[:, DIM:2 * DIM],
                out[:, 2 * DIM:3 * DIM], out[:, 3 * DIM:4 * DIM])

    x12, x21, x23, x32 = cross_pass(
        x_s1a, x_s2a, x_s3a,
        p["pool_s12_1"], p["unpool_s21_1"], p["pool_s23_1"], p["unpool_s32_1"])

    def tri_pass(arrays, scales, groups, l1, l2, l3):
        w = [wt(l1), bias(l1), wt(l2), bias(l2), wt(l3), bias(l3)]
        out = _fused_pass(
            Ab, arrays, scales, w, parts=groups,
            outspecs=[
                dict(terms=[(0, 0)], bias_id=1, act="relu", oscale_id=None,
                     col=0, wout=DIM),
                dict(terms=[(1, 2)], bias_id=3, act="relu", oscale_id=None,
                     col=DIM, wout=DIM),
                dict(terms=[(2, 4)], bias_id=5, act="relu", oscale_id=None,
                     col=2 * DIM, wout=DIM),
            ],
            score=None, c_out=3 * DIM)
        return out[:, 0:DIM], out[:, DIM:2 * DIM], out[:, 2 * DIM:3 * DIM]

    x_s1b, x_s2b, x_s3b = tri_pass(
        [x_s1a, x21, x_s1, x_s2a, x12, x32, x_s2, x_s3a, x23],
        [sel1, sel2, m2],
        [
            [(0, 1.0, None, None), (1, 1.0, None, None), (2, 1.0, None, None)],
            [(3, 1.0, 0, None), (4, 0.5, None, None), (5, 0.5, 0, None),
             (6, 1.0, 0, None)],
            [(7, 1.0, 1, None), (8, 1.0, None, None), (6, 1.0, 2, None)],
        ],
        p["s1_l2"], p["s2_l2"], p["s3_l2"])

    x12b, x21b, x23b, x32b = cross_pass(
        x_s1b, x_s2b, x_s3b,
        p["pool_s12_2"], p["unpool_s21_2"], p["pool_s23_2"], p["unpool_s32_2"])

    x_s1f, x_s2f, x_s3f = tri_pass(
        [x_s1b, x21b, x_s2b, x12b, x32b, x_s3b, x23b],
        [sel1, sel2],
        [
            [(0, 1.0, None, None), (1, 0.05, None, None)],
            [(2, 1.0, 0, None), (3, 0.025, None, None), (4, 0.025, 0, None)],
            [(5, 1.0, 1, None), (6, 0.05, None, None)],
        ],
        p["s1_l3"], p["s2_l3"], p["s3_l3"])

    u32e = _fused_pass(
        Ab, [x_s3f], [sel2],
        [wt(p["unpool_s32_end"]), bias(p["unpool_s32_end"])],
        parts=[[(0, 1.0, 0, None)]],
        outspecs=[dict(terms=[(0, 0)], bias_id=1, act=None,
                       oscale_id=None, col=0, wout=DIM)],
        score=None, c_out=DIM)

    x_s2out = _fused_pass(
        Ab, [x_s2f, u32e, xdown2], [sel1],
        [wt(p["unpool_s21_end"]), bias(p["unpool_s21_end"])],
        parts=[[(0, 1.0, 0, None), (1, 1.0, 0, None), (2, 1.0, 0, None)]],
        outspecs=[dict(terms=[(0, 0)], bias_id=1, act=None,
                       oscale_id=None, col=0, wout=DIM)],
        score=None, c_out=DIM)

    wend = wt(p["end_gcn"])
    out = _fused_pass(
        Ab, [x_s1f, x_s2out], [],
        [wend[0:DIM, :], wend[DIM:2 * DIM, :], bias(p["end_gcn"])],
        parts=[[(0, 1.0, None, None)], [(1, 1.0, None, None)]],
        outspecs=[dict(terms=[(0, 0), (1, 1)], bias_id=2, act=None,
                       oscale_id=None, col=0, wout=256)],
        score=None, c_out=256)
    return out


# speedup vs baseline: 1.3203x; 1.0767x over previous
import functools
from typing import Any

import jax
import jax.numpy as jnp
from jax import lax
from jax.experimental import pallas as pl
from jax.experimental.pallas import tpu as pltpu
from jax.experimental.pallas import tpu_sc as plsc

N = 4096
DIM = 48
K1 = int(0.8 * N)
K2 = int(0.7 * K1)
BM = 256
GRID = N // BM


def _fused_pass(A, arrays, scales, weights, parts, outspecs, score, c_out,
                emit_bf16=False):
    n_arr = len(arrays)
    n_sc = len(scales)

    def _entry_w(e):
        return (arrays[e[0]].shape[1] if e[3] is None
                else weights[e[3][0]].shape[1])

    group_w = [_entry_w(g[0]) for g in parts]
    c_in_total = sum(group_w)
    group_off = [sum(group_w[:gi]) for gi in range(len(parts))]

    def body(*refs):
        a_ref = refs[0]
        arr_refs = refs[1:1 + n_arr]
        sc_refs = refs[1 + n_arr:1 + n_arr + n_sc]
        w_refs = refs[1 + n_arr + n_sc:1 + n_arr + n_sc + len(weights)]
        x_scr = refs[-1]
        if emit_bf16:
            out_ref, ab_ref = refs[-3], refs[-2]
        else:
            out_ref = refs[-2]
        i = pl.program_id(0)

        def load_entry(entry, row_slice=None):
            aid, coef, sid, pre = entry
            v = (arr_refs[aid][...] if row_slice is None
                 else arr_refs[aid][row_slice, :])
            if pre is not None:
                wid, bid, act, a_id = pre
                v = jnp.dot(v, w_refs[wid][...],
                            preferred_element_type=jnp.float32)
                if bid is not None:
                    v = v + w_refs[bid][...]
                if act == "prelu":
                    a = w_refs[a_id][0, 0]
                    v = jnp.where(v >= 0.0, v, a * v)
            if sid is not None:
                s = (sc_refs[sid][...] if row_slice is None
                     else sc_refs[sid][row_slice, :])
                v = v * s
            if coef != 1.0:
                v = v * coef
            return v

        @pl.when(i == 0)
        def _assemble():
            for gi, group in enumerate(parts):
                acc = None
                for entry in group:
                    v = load_entry(entry)
                    acc = v if acc is None else acc + v
                x_scr[:, group_off[gi]:group_off[gi] + group_w[gi]] = (
                    acc.astype(x_scr.dtype))

        a_blk = a_ref[...]
        if emit_bf16:
            a_blk = a_blk.astype(jnp.bfloat16)
            ab_ref[...] = a_blk
        raws = []
        for gi in range(len(parts)):
            gv = x_scr[:, group_off[gi]:group_off[gi] + group_w[gi]]
            raws.append(jnp.dot(a_blk, gv,
                                preferred_element_type=jnp.float32))

        for spec in outspecs:
            y = None
            for (gi, wid) in spec["terms"]:
                t = raws[gi] if wid is None else jnp.dot(
                    raws[gi], w_refs[wid][...],
                    preferred_element_type=jnp.float32)
                y = t if y is None else y + t
            y = y + w_refs[spec["bias_id"]][...]
            if spec["act"] == "relu":
                y = jnp.maximum(y, 0.0)
            if spec["oscale_id"] is not None:
                y = y * sc_refs[spec["oscale_id"]][pl.ds(i * BM, BM), :]
            out_ref[:, spec["col"]:spec["col"] + spec["wout"]] = y

        if score is not None:
            hn = jnp.dot(raws[score["hn_group"]], w_refs[score["wg_id"]][...],
                         preferred_element_type=jnp.float32)
            hn = hn + w_refs[score["bg_id"]][...]
            if score["hn_col"] is not None:
                out_ref[:, score["hn_col"]:score["hn_col"] + DIM] = hn
            xs = jax.nn.sigmoid(hn)
            h = None
            for entry in parts[score["h_group"]]:
                v = load_entry(entry, row_slice=pl.ds(i * BM, BM))
                h = v if h is None else h + v
            hw = jnp.dot(h, w_refs[score["wd_id"]][...],
                         preferred_element_type=jnp.float32)
            t = jnp.sum(hw * xs, axis=1, keepdims=True)
            t = t + w_refs[score["bd_id"]][...]
            out_ref[:, score["col"]:score["col"] + 1] = jax.nn.sigmoid(t)

    in_specs = [pl.BlockSpec((BM, N), lambda i: (i, 0))]
    for a in arrays:
        w = a.shape[1]
        in_specs.append(pl.BlockSpec((N, w), lambda i: (0, 0)))
    for _ in scales:
        in_specs.append(pl.BlockSpec((N, 1), lambda i: (0, 0)))
    for wgt in weights:
        in_specs.append(pl.BlockSpec(wgt.shape, lambda i: (0, 0)))

    out_specs = pl.BlockSpec((BM, c_out), lambda i: (i, 0))
    out_shape = jax.ShapeDtypeStruct((N, c_out), jnp.float32)
    if emit_bf16:
        out_specs = [out_specs, pl.BlockSpec((BM, N), lambda i: (i, 0))]
        out_shape = [out_shape, jax.ShapeDtypeStruct((N, N), jnp.bfloat16)]

    return pl.pallas_call(
        body,
        grid=(GRID,),
        in_specs=in_specs,
        out_specs=out_specs,
        out_shape=out_shape,
        scratch_shapes=[pltpu.VMEM((N, c_in_total), jnp.bfloat16)],
    )(A, *arrays, *scales, *weights)


_BR = 128


def _rank_masks_tc(scores, valid, k):
    s_col = scores.reshape(N, 1)
    s_row = scores.reshape(1, N)
    use_valid = valid is not None

    def body(*refs):
        if use_valid:
            (sc_ref, sr_ref, vr_ref, vc_ref, m_ref, sel_ref) = refs
        else:
            (sc_ref, sr_ref, m_ref, sel_ref) = refs
        si = sc_ref[...]
        sj = sr_ref[...]
        gt = (sj > si).astype(jnp.float32)
        if use_valid:
            gt = gt * vr_ref[...]
        rank = jnp.sum(gt, axis=1, keepdims=True)
        sel = rank < float(k)
        if use_valid:
            sel = sel & (vc_ref[...] > 0.0)
        m_ref[...] = jnp.where(sel, si, 0.0)
        sel_ref[...] = jnp.where(sel, 1.0, 0.0)

    in_specs = [pl.BlockSpec((_BR, 1), lambda i: (i, 0)),
                pl.BlockSpec((1, N), lambda i: (0, 0))]
    args = [s_col, s_row]
    if use_valid:
        in_specs.append(pl.BlockSpec((1, N), lambda i: (0, 0)))
        in_specs.append(pl.BlockSpec((_BR, 1), lambda i: (i, 0)))
        args.append(valid.reshape(1, N))
        args.append(valid.reshape(N, 1))

    return pl.pallas_call(
        body,
        grid=(N // _BR,),
        in_specs=in_specs,
        out_specs=[pl.BlockSpec((_BR, 1), lambda i: (i, 0)),
                   pl.BlockSpec((_BR, 1), lambda i: (i, 0))],
        out_shape=[jax.ShapeDtypeStruct((N, 1), jnp.float32),
                   jax.ShapeDtypeStruct((N, 1), jnp.float32)],
    )(*args)


_NV = N // 16


def _rank_masks_sc(scores, valid, k):
    use_valid = valid is not None
    mesh = plsc.VectorSubcoreMesh(core_axis_name="c", subcore_axis_name="s")
    n_in = 2 if use_valid else 1

    @functools.partial(
        pl.kernel, mesh=mesh,
        out_type=[jax.ShapeDtypeStruct((N,), jnp.float32),
                  jax.ShapeDtypeStruct((N,), jnp.float32)],
        scratch_types=[pltpu.VMEM((N + 32,), jnp.float32),
                       pltpu.VMEM((N,), jnp.float32),
                       pltpu.VMEM((128,), jnp.float32),
                       pltpu.VMEM((128,), jnp.float32)],
    )
    def rank_kernel(*refs):
        s_hbm = refs[0]
        v_hbm = refs[1] if use_valid else None
        m_hbm, sel_hbm = refs[n_in], refs[n_in + 1]
        sm_v, v_v, m_loc, sel_loc = refs[n_in + 2:n_in + 6]

        wid = lax.axis_index("s") * 2 + lax.axis_index("c")
        base = wid * 128
        neg = jnp.full((16,), -1.0, jnp.float32)
        pltpu.sync_copy(s_hbm, sm_v.at[pl.ds(16, N)])
        sm_v[pl.ds(0, 16)] = neg
        sm_v[pl.ds(N + 16, 16)] = neg
        if use_valid:
            pltpu.sync_copy(v_hbm, v_v)
            for q in range(_NV):
                sl = pl.ds(16 + q * 16, 16)
                sm_v[sl] = jnp.where(v_v[pl.ds(q * 16, 16)] > 0.0,
                                     sm_v[sl], -1.0)

        mines = [sm_v[pl.ds(16 + base + e * 16, 16)] for e in range(8)]

        NB = 4

        def jbody(j, carry):
            banks = [list(b) for b in carry]
            jb = j * 64
            for r0 in range(0, 64, NB):
                ws = [sm_v[pl.ds(16 + jb + r0 + q, 16)] for q in range(NB)]
                for e in range(8):
                    for q in range(NB):
                        banks[q][e] = jnp.where(ws[q] > mines[e],
                                                banks[q][e] + 1.0,
                                                banks[q][e])
            return tuple(tuple(b) for b in banks)

        zero = jnp.zeros((16,), jnp.float32)
        init = tuple((zero,) * 8 for _ in range(NB))
        banks = lax.fori_loop(0, _NV // 4, jbody, init)
        accs = [banks[0][e] + banks[1][e] + banks[2][e] + banks[3][e]
                for e in range(8)]

        for d in range(1, 16):
            w = sm_v[pl.ds(16 - d, 16)]
            for e in range(8):
                accs[e] = accs[e] + jnp.where(w > mines[e], 1.0, 0.0)

        kf = jnp.float32(k)
        for e in range(8):
            sel_e = jnp.where(accs[e] < kf, 1.0, 0.0)
            sl = pl.ds(e * 16, 16)
            sel_loc[sl] = sel_e
            m_loc[sl] = sel_e * mines[e]
        pltpu.sync_copy(m_loc, m_hbm.at[pl.ds(base, 128)])
        pltpu.sync_copy(sel_loc, sel_hbm.at[pl.ds(base, 128)])

    args = (scores, valid) if use_valid else (scores,)
    m, sel = rank_kernel(*args)
    return m.reshape(N, 1), sel.reshape(N, 1)


def kernel(A, x, params: dict[str, Any]):
    p = params

    def wt(lin):
        return lin["W"].T

    def bias(lin):
        return lin["b"].reshape(1, -1)

    x_s1, Ab = _fused_pass(
        A, [x], [],
        [wt(p["start_gcn_s1"]), bias(p["start_gcn_s1"])],
        parts=[[(0, 1.0, None, (0, None, None, None))]],
        outspecs=[dict(terms=[(0, None)], bias_id=1, act=None,
                       oscale_id=None, col=0, wout=DIM)],
        score=None, c_out=DIM, emit_bf16=True)

    is1 = p["is1"]
    w2 = [wt(p["s1_l1"]), bias(p["s1_l1"]),
          wt(is1["gcn1"]), bias(is1["gcn1"]),
          is1["disc"]["W"][0], is1["disc"]["b"].reshape(1, 1),
          wt(is1["fc"]), (is1["fc"]["b"] + is1["fc"]["bias2"]).reshape(1, -1),
          is1["fc"]["a"].reshape(1, 1)]
    pass2 = _fused_pass(
        Ab, [x_s1], [], w2,
        parts=[[(0, 1.0, None, (6, 7, "prelu", 8))], [(0, 1.0, None, None)]],
        outspecs=[dict(terms=[(1, 0)], bias_id=1, act="relu",
                       oscale_id=None, col=0, wout=DIM)],
        score=dict(hn_group=0, wg_id=2, bg_id=3, wd_id=4, bd_id=5,
                   h_group=0, col=DIM, hn_col=None),
        c_out=DIM + 1)
    x_s1a = pass2[:, 0:DIM]
    scores1 = pass2[:, DIM]
    m1, sel1 = _rank_masks_sc(scores1, None, K1)

    x_s2 = _fused_pass(
        Ab, [x_s1], [m1], [wt(p["start_gcn_s2"]), bias(p["start_gcn_s2"])],
        parts=[[(0, 1.0, 0, None)]],
        outspecs=[dict(terms=[(0, 0)], bias_id=1, act=None,
                       oscale_id=None, col=0, wout=DIM)],
        score=None, c_out=DIM)

    is2 = p["is2"]
    w4 = [wt(p["s2_l1"]), bias(p["s2_l1"]),
          wt(is2["gcn1"]), bias(is2["gcn1"]),
          is2["disc"]["W"][0], is2["disc"]["b"].reshape(1, 1),
          wt(is2["fc"]), (is2["fc"]["b"] + is2["fc"]["bias2"]).reshape(1, -1),
          is2["fc"]["a"].reshape(1, 1)]
    pass4 = _fused_pass(
        Ab, [x_s2], [sel1], w4,
        parts=[[(0, 1.0, 0, (6, 7, "prelu", 8))], [(0, 1.0, 0, None)]],
        outspecs=[dict(terms=[(1, 0)], bias_id=1, act="relu",
                       oscale_id=None, col=DIM, wout=DIM)],
        score=dict(hn_group=0, wg_id=2, bg_id=3, wd_id=4, bd_id=5,
                   h_group=0, col=2 * DIM, hn_col=0),
        c_out=2 * DIM + 1)
    xdown2 = pass4[:, 0:DIM]
    x_s2a = pass4[:, DIM:2 * DIM]
    scores2 = pass4[:, 2 * DIM]
    m2, sel2 = _rank_masks_sc(scores2, sel1.reshape(N), K2)

    x_s3a = _fused_pass(
        Ab, [x_s2], [m2], [wt(p["s3_l1"]), bias(p["s3_l1"])],
        parts=[[(0, 1.0, 0, None)]],
        outspecs=[dict(terms=[(0, 0)], bias_id=1, act="relu",
                       oscale_id=None, col=0, wout=DIM)],
        score=None, c_out=DIM)

    def cross_pass(xs1, xs2, xs3, wp12, wu21, wp23, wu32):
        w = [wt(wp12), bias(wp12), wt(wu21), bias(wu21),
             wt(wp23), bias(wp23), wt(wu32), bias(wu32)]
        out = _fused_pass(
            Ab, [xs1, xs2, xs3], [sel1, sel2, m1, m2], w,
            parts=[[(0, 1.0, None, None)], [(1, 1.0, 0, None)],
                   [(2, 1.0, 1, None)]],
            outspecs=[
                dict(terms=[(0, 0)], bias_id=1, act=None, oscale_id=2,
                     col=0, wout=DIM),
                dict(terms=[(1, 2)], bias_id=3, act=None, oscale_id=None,
                     col=DIM, wout=DIM),
                dict(terms=[(1, 4)], bias_id=5, act=None, oscale_id=3,
                     col=2 * DIM, wout=DIM),
                dict(terms=[(2, 6)], bias_id=7, act=None, oscale_id=None,
                     col=3 * DIM, wout=DIM),
            ],
            score=None, c_out=4 * DIM)
        return (out[:, 0:DIM], out[:, DIM:2 * DIM],
                out[:, 2 * DIM:3 * DIM], out[:, 3 * DIM:4 * DIM])

    x12, x21, x23, x32 = cross_pass(
        x_s1a, x_s2a, x_s3a,
        p["pool_s12_1"], p["unpool_s21_1"], p["pool_s23_1"], p["unpool_s32_1"])

    def tri_pass(arrays, scales, groups, l1, l2, l3):
        w = [wt(l1), bias(l1), wt(l2), bias(l2), wt(l3), bias(l3)]
        out = _fused_pass(
            Ab, arrays, scales, w, parts=groups,
            outspecs=[
                dict(terms=[(0, 0)], bias_id=1, act="relu", oscale_id=None,
                     col=0, wout=DIM),
                dict(terms=[(1, 2)], bias_id=3, act="relu", oscale_id=None,
                     col=DIM, wout=DIM),
                dict(terms=[(2, 4)], bias_id=5, act="relu", oscale_id=None,
                     col=2 * DIM, wout=DIM),
            ],
            score=None, c_out=3 * DIM)
        return out[:, 0:DIM], out[:, DIM:2 * DIM], out[:, 2 * DIM:3 * DIM]

    x_s1b, x_s2b, x_s3b = tri_pass(
        [x_s1a, x21, x_s1, x_s2a, x12, x32, x_s2, x_s3a, x23],
        [sel1, sel2, m2],
        [
            [(0, 1.0, None, None), (1, 1.0, None, None), (2, 1.0, None, None)],
            [(3, 1.0, 0, None), (4, 0.5, None, None), (5, 0.5, 0, None),
             (6, 1.0, 0, None)],
            [(7, 1.0, 1, None), (8, 1.0, None, None), (6, 1.0, 2, None)],
        ],
        p["s1_l2"], p["s2_l2"], p["s3_l2"])

    x12b, x21b, x23b, x32b = cross_pass(
        x_s1b, x_s2b, x_s3b,
        p["pool_s12_2"], p["unpool_s21_2"], p["pool_s23_2"], p["unpool_s32_2"])

    x_s1f, x_s2f, x_s3f = tri_pass(
        [x_s1b, x21b, x_s2b, x12b, x32b, x_s3b, x23b],
        [sel1, sel2],
        [
            [(0, 1.0, None, None), (1, 0.05, None, None)],
            [(2, 1.0, 0, None), (3, 0.025, None, None), (4, 0.025, 0, None)],
            [(5, 1.0, 1, None), (6, 0.05, None, None)],
        ],
        p["s1_l3"], p["s2_l3"], p["s3_l3"])

    u32e = _fused_pass(
        Ab, [x_s3f], [sel2],
        [wt(p["unpool_s32_end"]), bias(p["unpool_s32_end"])],
        parts=[[(0, 1.0, 0, None)]],
        outspecs=[dict(terms=[(0, 0)], bias_id=1, act=None,
                       oscale_id=None, col=0, wout=DIM)],
        score=None, c_out=DIM)

    x_s2out = _fused_pass(
        Ab, [x_s2f, u32e, xdown2], [sel1],
        [wt(p["unpool_s21_end"]), bias(p["unpool_s21_end"])],
        parts=[[(0, 1.0, 0, None), (1, 1.0, 0, None), (2, 1.0, 0, None)]],
        outspecs=[dict(terms=[(0, 0)], bias_id=1, act=None,
                       oscale_id=None, col=0, wout=DIM)],
        score=None, c_out=DIM)

    wend = wt(p["end_gcn"])
    out = _fused_pass(
        Ab, [x_s1f, x_s2out], [],
        [wend[0:DIM, :], wend[DIM:2 * DIM, :], bias(p["end_gcn"])],
        parts=[[(0, 1.0, None, None)], [(1, 1.0, None, None)]],
        outspecs=[dict(terms=[(0, 0), (1, 1)], bias_id=2, act=None,
                       oscale_id=None, col=0, wout=256)],
        score=None, c_out=256)
    return out


# speedup vs baseline: 1.4244x; 1.0789x over previous
import functools
from typing import Any

import jax
import jax.numpy as jnp
from jax import lax
from jax.experimental import pallas as pl
from jax.experimental.pallas import tpu as pltpu
from jax.experimental.pallas import tpu_sc as plsc

N = 4096
DIM = 48
K1 = int(0.8 * N)
K2 = int(0.7 * K1)
BM = 512
GRID = N // BM


def _fused_pass(A, arrays, scales, weights, parts, outspecs, score, c_out,
                emit_bf16=False):
    n_arr = len(arrays)
    n_sc = len(scales)

    def _entry_w(e):
        return (arrays[e[0]].shape[1] if e[3] is None
                else weights[e[3][0]].shape[1])

    group_w = [_entry_w(g[0]) for g in parts]
    c_in_total = sum(group_w)
    group_off = [sum(group_w[:gi]) for gi in range(len(parts))]

    def body(*refs):
        a_ref = refs[0]
        arr_refs = refs[1:1 + n_arr]
        sc_refs = refs[1 + n_arr:1 + n_arr + n_sc]
        w_refs = refs[1 + n_arr + n_sc:1 + n_arr + n_sc + len(weights)]
        x_scr = refs[-1]
        if emit_bf16:
            out_ref, ab_ref = refs[-3], refs[-2]
        else:
            out_ref = refs[-2]
        i = pl.program_id(0)

        def load_entry(entry, row_slice=None):
            aid, coef, sid, pre = entry
            v = (arr_refs[aid][...] if row_slice is None
                 else arr_refs[aid][row_slice, :])
            if pre is not None:
                wid, bid, act, a_id = pre
                v = jnp.dot(v, w_refs[wid][...],
                            preferred_element_type=jnp.float32)
                if bid is not None:
                    v = v + w_refs[bid][...]
                if act == "prelu":
                    a = w_refs[a_id][0, 0]
                    v = jnp.where(v >= 0.0, v, a * v)
            if sid is not None:
                s = (sc_refs[sid][...] if row_slice is None
                     else sc_refs[sid][row_slice, :])
                v = v * s
            if coef != 1.0:
                v = v * coef
            return v

        @pl.when(i == 0)
        def _assemble():
            for gi, group in enumerate(parts):
                acc = None
                for entry in group:
                    v = load_entry(entry)
                    acc = v if acc is None else acc + v
                x_scr[:, group_off[gi]:group_off[gi] + group_w[gi]] = (
                    acc.astype(x_scr.dtype))

        a_blk = a_ref[...]
        if emit_bf16:
            a_blk = a_blk.astype(jnp.bfloat16)
            ab_ref[...] = a_blk
        raws = []
        for gi in range(len(parts)):
            gv = x_scr[:, group_off[gi]:group_off[gi] + group_w[gi]]
            raws.append(jnp.dot(a_blk, gv,
                                preferred_element_type=jnp.float32))

        for spec in outspecs:
            y = None
            for (gi, wid) in spec["terms"]:
                t = raws[gi] if wid is None else jnp.dot(
                    raws[gi], w_refs[wid][...],
                    preferred_element_type=jnp.float32)
                y = t if y is None else y + t
            y = y + w_refs[spec["bias_id"]][...]
            if spec["act"] == "relu":
                y = jnp.maximum(y, 0.0)
            if spec["oscale_id"] is not None:
                y = y * sc_refs[spec["oscale_id"]][pl.ds(i * BM, BM), :]
            out_ref[:, spec["col"]:spec["col"] + spec["wout"]] = y

        if score is not None:
            hn = jnp.dot(raws[score["hn_group"]], w_refs[score["wg_id"]][...],
                         preferred_element_type=jnp.float32)
            hn = hn + w_refs[score["bg_id"]][...]
            if score["hn_col"] is not None:
                out_ref[:, score["hn_col"]:score["hn_col"] + DIM] = hn
            xs = jax.nn.sigmoid(hn)
            h = None
            for entry in parts[score["h_group"]]:
                v = load_entry(entry, row_slice=pl.ds(i * BM, BM))
                h = v if h is None else h + v
            hw = jnp.dot(h, w_refs[score["wd_id"]][...],
                         preferred_element_type=jnp.float32)
            t = jnp.sum(hw * xs, axis=1, keepdims=True)
            t = t + w_refs[score["bd_id"]][...]
            out_ref[:, score["col"]:score["col"] + 1] = jax.nn.sigmoid(t)

    in_specs = [pl.BlockSpec((BM, N), lambda i: (i, 0))]
    for a in arrays:
        w = a.shape[1]
        in_specs.append(pl.BlockSpec((N, w), lambda i: (0, 0)))
    for _ in scales:
        in_specs.append(pl.BlockSpec((N, 1), lambda i: (0, 0)))
    for wgt in weights:
        in_specs.append(pl.BlockSpec(wgt.shape, lambda i: (0, 0)))

    out_specs = pl.BlockSpec((BM, c_out), lambda i: (i, 0))
    out_shape = jax.ShapeDtypeStruct((N, c_out), jnp.float32)
    if emit_bf16:
        out_specs = [out_specs, pl.BlockSpec((BM, N), lambda i: (i, 0))]
        out_shape = [out_shape, jax.ShapeDtypeStruct((N, N), jnp.bfloat16)]

    return pl.pallas_call(
        body,
        grid=(GRID,),
        in_specs=in_specs,
        out_specs=out_specs,
        out_shape=out_shape,
        scratch_shapes=[pltpu.VMEM((N, c_in_total), jnp.bfloat16)],
    )(A, *arrays, *scales, *weights)


_BR = 128


def _rank_masks_tc(scores, valid, k):
    s_col = scores.reshape(N, 1)
    s_row = scores.reshape(1, N)
    use_valid = valid is not None

    def body(*refs):
        if use_valid:
            (sc_ref, sr_ref, vr_ref, vc_ref, m_ref, sel_ref) = refs
        else:
            (sc_ref, sr_ref, m_ref, sel_ref) = refs
        si = sc_ref[...]
        sj = sr_ref[...]
        gt = (sj > si).astype(jnp.float32)
        if use_valid:
            gt = gt * vr_ref[...]
        rank = jnp.sum(gt, axis=1, keepdims=True)
        sel = rank < float(k)
        if use_valid:
            sel = sel & (vc_ref[...] > 0.0)
        m_ref[...] = jnp.where(sel, si, 0.0)
        sel_ref[...] = jnp.where(sel, 1.0, 0.0)

    in_specs = [pl.BlockSpec((_BR, 1), lambda i: (i, 0)),
                pl.BlockSpec((1, N), lambda i: (0, 0))]
    args = [s_col, s_row]
    if use_valid:
        in_specs.append(pl.BlockSpec((1, N), lambda i: (0, 0)))
        in_specs.append(pl.BlockSpec((_BR, 1), lambda i: (i, 0)))
        args.append(valid.reshape(1, N))
        args.append(valid.reshape(N, 1))

    return pl.pallas_call(
        body,
        grid=(N // _BR,),
        in_specs=in_specs,
        out_specs=[pl.BlockSpec((_BR, 1), lambda i: (i, 0)),
                   pl.BlockSpec((_BR, 1), lambda i: (i, 0))],
        out_shape=[jax.ShapeDtypeStruct((N, 1), jnp.float32),
                   jax.ShapeDtypeStruct((N, 1), jnp.float32)],
    )(*args)


_NV = N // 16


def _rank_masks_sc(scores, valid, k):
    use_valid = valid is not None
    mesh = plsc.VectorSubcoreMesh(core_axis_name="c", subcore_axis_name="s")
    n_in = 2 if use_valid else 1

    @functools.partial(
        pl.kernel, mesh=mesh,
        out_type=[jax.ShapeDtypeStruct((N,), jnp.float32),
                  jax.ShapeDtypeStruct((N,), jnp.float32)],
        scratch_types=[pltpu.VMEM((N + 32,), jnp.float32),
                       pltpu.VMEM((N,), jnp.float32),
                       pltpu.VMEM((128,), jnp.float32),
                       pltpu.VMEM((128,), jnp.float32)],
    )
    def rank_kernel(*refs):
        s_hbm = refs[0]
        v_hbm = refs[1] if use_valid else None
        m_hbm, sel_hbm = refs[n_in], refs[n_in + 1]
        sm_v, v_v, m_loc, sel_loc = refs[n_in + 2:n_in + 6]

        wid = lax.axis_index("s") * 2 + lax.axis_index("c")
        base = wid * 128
        neg = jnp.full((16,), -1.0, jnp.float32)
        pltpu.sync_copy(s_hbm, sm_v.at[pl.ds(16, N)])
        sm_v[pl.ds(0, 16)] = neg
        sm_v[pl.ds(N + 16, 16)] = neg
        if use_valid:
            pltpu.sync_copy(v_hbm, v_v)
            for q in range(_NV):
                sl = pl.ds(16 + q * 16, 16)
                sm_v[sl] = jnp.where(v_v[pl.ds(q * 16, 16)] > 0.0,
                                     sm_v[sl], -1.0)

        mines = [sm_v[pl.ds(16 + base + e * 16, 16)] for e in range(8)]

        NB = 4

        def jbody(j, carry):
            banks = [list(b) for b in carry]
            jb = j * 64
            for r0 in range(0, 64, NB):
                ws = [sm_v[pl.ds(16 + jb + r0 + q, 16)] for q in range(NB)]
                for e in range(8):
                    for q in range(NB):
                        banks[q][e] = jnp.where(ws[q] > mines[e],
                                                banks[q][e] + 1.0,
                                                banks[q][e])
            return tuple(tuple(b) for b in banks)

        zero = jnp.zeros((16,), jnp.float32)
        init = tuple((zero,) * 8 for _ in range(NB))
        banks = lax.fori_loop(0, _NV // 4, jbody, init)
        accs = [banks[0][e] + banks[1][e] + banks[2][e] + banks[3][e]
                for e in range(8)]

        for d in range(1, 16):
            w = sm_v[pl.ds(16 - d, 16)]
            for e in range(8):
                accs[e] = accs[e] + jnp.where(w > mines[e], 1.0, 0.0)

        kf = jnp.float32(k)
        for e in range(8):
            sel_e = jnp.where(accs[e] < kf, 1.0, 0.0)
            sl = pl.ds(e * 16, 16)
            sel_loc[sl] = sel_e
            m_loc[sl] = sel_e * mines[e]
        pltpu.sync_copy(m_loc, m_hbm.at[pl.ds(base, 128)])
        pltpu.sync_copy(sel_loc, sel_hbm.at[pl.ds(base, 128)])

    args = (scores, valid) if use_valid else (scores,)
    m, sel = rank_kernel(*args)
    return m.reshape(N, 1), sel.reshape(N, 1)


def kernel(A, x, params: dict[str, Any]):
    p = params

    def wt(lin):
        return lin["W"].T

    def bias(lin):
        return lin["b"].reshape(1, -1)

    x_s1, Ab = _fused_pass(
        A, [x], [],
        [wt(p["start_gcn_s1"]), bias(p["start_gcn_s1"])],
        parts=[[(0, 1.0, None, (0, None, None, None))]],
        outspecs=[dict(terms=[(0, None)], bias_id=1, act=None,
                       oscale_id=None, col=0, wout=DIM)],
        score=None, c_out=DIM, emit_bf16=True)

    is1 = p["is1"]
    w2 = [wt(p["s1_l1"]), bias(p["s1_l1"]),
          wt(is1["gcn1"]), bias(is1["gcn1"]),
          is1["disc"]["W"][0], is1["disc"]["b"].reshape(1, 1),
          wt(is1["fc"]), (is1["fc"]["b"] + is1["fc"]["bias2"]).reshape(1, -1),
          is1["fc"]["a"].reshape(1, 1)]
    pass2 = _fused_pass(
        Ab, [x_s1], [], w2,
        parts=[[(0, 1.0, None, (6, 7, "prelu", 8))], [(0, 1.0, None, None)]],
        outspecs=[dict(terms=[(1, 0)], bias_id=1, act="relu",
                       oscale_id=None, col=0, wout=DIM)],
        score=dict(hn_group=0, wg_id=2, bg_id=3, wd_id=4, bd_id=5,
                   h_group=0, col=DIM, hn_col=None),
        c_out=DIM + 1)
    x_s1a = pass2[:, 0:DIM]
    scores1 = pass2[:, DIM]
    m1, sel1 = _rank_masks_sc(scores1, None, K1)

    x_s2 = _fused_pass(
        Ab, [x_s1], [m1], [wt(p["start_gcn_s2"]), bias(p["start_gcn_s2"])],
        parts=[[(0, 1.0, 0, None)]],
        outspecs=[dict(terms=[(0, 0)], bias_id=1, act=None,
                       oscale_id=None, col=0, wout=DIM)],
        score=None, c_out=DIM)

    is2 = p["is2"]
    w4 = [wt(p["s2_l1"]), bias(p["s2_l1"]),
          wt(is2["gcn1"]), bias(is2["gcn1"]),
          is2["disc"]["W"][0], is2["disc"]["b"].reshape(1, 1),
          wt(is2["fc"]), (is2["fc"]["b"] + is2["fc"]["bias2"]).reshape(1, -1),
          is2["fc"]["a"].reshape(1, 1)]
    pass4 = _fused_pass(
        Ab, [x_s2], [sel1], w4,
        parts=[[(0, 1.0, 0, (6, 7, "prelu", 8))], [(0, 1.0, 0, None)]],
        outspecs=[dict(terms=[(1, 0)], bias_id=1, act="relu",
                       oscale_id=None, col=DIM, wout=DIM)],
        score=dict(hn_group=0, wg_id=2, bg_id=3, wd_id=4, bd_id=5,
                   h_group=0, col=2 * DIM, hn_col=0),
        c_out=2 * DIM + 1)
    xdown2 = pass4[:, 0:DIM]
    x_s2a = pass4[:, DIM:2 * DIM]
    scores2 = pass4[:, 2 * DIM]
    m2, sel2 = _rank_masks_sc(scores2, sel1.reshape(N), K2)

    x_s3a = _fused_pass(
        Ab, [x_s2], [m2], [wt(p["s3_l1"]), bias(p["s3_l1"])],
        parts=[[(0, 1.0, 0, None)]],
        outspecs=[dict(terms=[(0, 0)], bias_id=1, act="relu",
                       oscale_id=None, col=0, wout=DIM)],
        score=None, c_out=DIM)

    def cross_pass(xs1, xs2, xs3, wp12, wu21, wp23, wu32):
        w = [wt(wp12), bias(wp12), wt(wu21), bias(wu21),
             wt(wp23), bias(wp23), wt(wu32), bias(wu32)]
        out = _fused_pass(
            Ab, [xs1, xs2, xs3], [sel1, sel2, m1, m2], w,
            parts=[[(0, 1.0, None, None)], [(1, 1.0, 0, None)],
                   [(2, 1.0, 1, None)]],
            outspecs=[
                dict(terms=[(0, 0)], bias_id=1, act=None, oscale_id=2,
                     col=0, wout=DIM),
                dict(terms=[(1, 2)], bias_id=3, act=None, oscale_id=None,
                     col=DIM, wout=DIM),
                dict(terms=[(1, 4)], bias_id=5, act=None, oscale_id=3,
                     col=2 * DIM, wout=DIM),
                dict(terms=[(2, 6)], bias_id=7, act=None, oscale_id=None,
                     col=3 * DIM, wout=DIM),
            ],
            score=None, c_out=4 * DIM)
        return (out[:, 0:DIM], out[:, DIM:2 * DIM],
                out[:, 2 * DIM:3 * DIM], out[:, 3 * DIM:4 * DIM])

    x12, x21, x23, x32 = cross_pass(
        x_s1a, x_s2a, x_s3a,
        p["pool_s12_1"], p["unpool_s21_1"], p["pool_s23_1"], p["unpool_s32_1"])

    def tri_pass(arrays, scales, groups, l1, l2, l3):
        w = [wt(l1), bias(l1), wt(l2), bias(l2), wt(l3), bias(l3)]
        out = _fused_pass(
            Ab, arrays, scales, w, parts=groups,
            outspecs=[
                dict(terms=[(0, 0)], bias_id=1, act="relu", oscale_id=None,
                     col=0, wout=DIM),
                dict(terms=[(1, 2)], bias_id=3, act="relu", oscale_id=None,
                     col=DIM, wout=DIM),
                dict(terms=[(2, 4)], bias_id=5, act="relu", oscale_id=None,
                     col=2 * DIM, wout=DIM),
            ],
            score=None, c_out=3 * DIM)
        return out[:, 0:DIM], out[:, DIM:2 * DIM], out[:, 2 * DIM:3 * DIM]

    x_s1b, x_s2b, x_s3b = tri_pass(
        [x_s1a, x21, x_s1, x_s2a, x12, x32, x_s2, x_s3a, x23],
        [sel1, sel2, m2],
        [
            [(0, 1.0, None, None), (1, 1.0, None, None), (2, 1.0, None, None)],
            [(3, 1.0, 0, None), (4, 0.5, None, None), (5, 0.5, 0, None),
             (6, 1.0, 0, None)],
            [(7, 1.0, 1, None), (8, 1.0, None, None), (6, 1.0, 2, None)],
        ],
        p["s1_l2"], p["s2_l2"], p["s3_l2"])

    x12b, x21b, x23b, x32b = cross_pass(
        x_s1b, x_s2b, x_s3b,
        p["pool_s12_2"], p["unpool_s21_2"], p["pool_s23_2"], p["unpool_s32_2"])

    x_s1f, x_s2f, x_s3f = tri_pass(
        [x_s1b, x21b, x_s2b, x12b, x32b, x_s3b, x23b],
        [sel1, sel2],
        [
            [(0, 1.0, None, None), (1, 0.05, None, None)],
            [(2, 1.0, 0, None), (3, 0.025, None, None), (4, 0.025, 0, None)],
            [(5, 1.0, 1, None), (6, 0.05, None, None)],
        ],
        p["s1_l3"], p["s2_l3"], p["s3_l3"])

    u32e = _fused_pass(
        Ab, [x_s3f], [sel2],
        [wt(p["unpool_s32_end"]), bias(p["unpool_s32_end"])],
        parts=[[(0, 1.0, 0, None)]],
        outspecs=[dict(terms=[(0, 0)], bias_id=1, act=None,
                       oscale_id=None, col=0, wout=DIM)],
        score=None, c_out=DIM)

    x_s2out = _fused_pass(
        Ab, [x_s2f, u32e, xdown2], [sel1],
        [wt(p["unpool_s21_end"]), bias(p["unpool_s21_end"])],
        parts=[[(0, 1.0, 0, None), (1, 1.0, 0, None), (2, 1.0, 0, None)]],
        outspecs=[dict(terms=[(0, 0)], bias_id=1, act=None,
                       oscale_id=None, col=0, wout=DIM)],
        score=None, c_out=DIM)

    wend = wt(p["end_gcn"])
    out = _fused_pass(
        Ab, [x_s1f, x_s2out], [],
        [wend[0:DIM, :], wend[DIM:2 * DIM, :], bias(p["end_gcn"])],
        parts=[[(0, 1.0, None, None)], [(1, 1.0, None, None)]],
        outspecs=[dict(terms=[(0, 0), (1, 1)], bias_id=2, act=None,
                       oscale_id=None, col=0, wout=256)],
        score=None, c_out=256)
    return out


# speedup vs baseline: 1.4253x; 1.0006x over previous
import functools
from typing import Any

import jax
import jax.numpy as jnp
from jax import lax
from jax.experimental import pallas as pl
from jax.experimental.pallas import tpu as pltpu
from jax.experimental.pallas import tpu_sc as plsc

N = 4096
DIM = 48
K1 = int(0.8 * N)
K2 = int(0.7 * K1)
BM = 512
GRID = N // BM


def _fused_pass(A, arrays, scales, weights, parts, outspecs, score, c_out,
                emit_bf16=False):
    n_arr = len(arrays)
    n_sc = len(scales)

    def _entry_w(e):
        return (arrays[e[0]].shape[1] if e[3] is None
                else weights[e[3][0]].shape[1])

    group_w = [_entry_w(g[0]) for g in parts]
    c_in_total = sum(group_w)
    group_off = [sum(group_w[:gi]) for gi in range(len(parts))]

    def body(*refs):
        a_ref = refs[0]
        arr_refs = refs[1:1 + n_arr]
        sc_refs = refs[1 + n_arr:1 + n_arr + n_sc]
        w_refs = refs[1 + n_arr + n_sc:1 + n_arr + n_sc + len(weights)]
        x_scr = refs[-1]
        if emit_bf16:
            out_ref, ab_ref = refs[-3], refs[-2]
        else:
            out_ref = refs[-2]
        i = pl.program_id(0)

        def load_entry(entry, row_slice=None):
            aid, coef, sid, pre = entry
            v = (arr_refs[aid][...] if row_slice is None
                 else arr_refs[aid][row_slice, :])
            if pre is not None:
                wid, bid, act, a_id = pre
                v = jnp.dot(v, w_refs[wid][...],
                            preferred_element_type=jnp.float32)
                if bid is not None:
                    v = v + w_refs[bid][...]
                if act == "prelu":
                    a = w_refs[a_id][0, 0]
                    v = jnp.where(v >= 0.0, v, a * v)
            if sid is not None:
                s = (sc_refs[sid][...] if row_slice is None
                     else sc_refs[sid][row_slice, :])
                v = v * s
            if coef != 1.0:
                v = v * coef
            return v

        @pl.when(i == 0)
        def _assemble():
            for gi, group in enumerate(parts):
                acc = None
                for entry in group:
                    v = load_entry(entry)
                    acc = v if acc is None else acc + v
                x_scr[:, group_off[gi]:group_off[gi] + group_w[gi]] = (
                    acc.astype(x_scr.dtype))

        a_blk = a_ref[...]
        if emit_bf16:
            a_blk = a_blk.astype(jnp.bfloat16)
            ab_ref[...] = a_blk
        raws = []
        for gi in range(len(parts)):
            gv = x_scr[:, group_off[gi]:group_off[gi] + group_w[gi]]
            raws.append(jnp.dot(a_blk, gv,
                                preferred_element_type=jnp.float32))

        for spec in outspecs:
            y = None
            for (gi, wid) in spec["terms"]:
                t = raws[gi] if wid is None else jnp.dot(
                    raws[gi], w_refs[wid][...],
                    preferred_element_type=jnp.float32)
                y = t if y is None else y + t
            y = y + w_refs[spec["bias_id"]][...]
            if spec["act"] == "relu":
                y = jnp.maximum(y, 0.0)
            if spec["oscale_id"] is not None:
                y = y * sc_refs[spec["oscale_id"]][pl.ds(i * BM, BM), :]
            out_ref[:, spec["col"]:spec["col"] + spec["wout"]] = y

        if score is not None:
            hn = jnp.dot(raws[score["hn_group"]], w_refs[score["wg_id"]][...],
                         preferred_element_type=jnp.float32)
            hn = hn + w_refs[score["bg_id"]][...]
            if score["hn_col"] is not None:
                out_ref[:, score["hn_col"]:score["hn_col"] + DIM] = hn
            xs = jax.nn.sigmoid(hn)
            h = None
            for entry in parts[score["h_group"]]:
                v = load_entry(entry, row_slice=pl.ds(i * BM, BM))
                h = v if h is None else h + v
            hw = jnp.dot(h, w_refs[score["wd_id"]][...],
                         preferred_element_type=jnp.float32)
            t = jnp.sum(hw * xs, axis=1, keepdims=True)
            t = t + w_refs[score["bd_id"]][...]
            out_ref[:, score["col"]:score["col"] + 1] = jax.nn.sigmoid(t)

    in_specs = [pl.BlockSpec((BM, N), lambda i: (i, 0))]
    for a in arrays:
        w = a.shape[1]
        in_specs.append(pl.BlockSpec((N, w), lambda i: (0, 0)))
    for _ in scales:
        in_specs.append(pl.BlockSpec((N, 1), lambda i: (0, 0)))
    for wgt in weights:
        in_specs.append(pl.BlockSpec(wgt.shape, lambda i: (0, 0)))

    out_specs = pl.BlockSpec((BM, c_out), lambda i: (i, 0))
    out_shape = jax.ShapeDtypeStruct((N, c_out), jnp.float32)
    if emit_bf16:
        out_specs = [out_specs, pl.BlockSpec((BM, N), lambda i: (i, 0))]
        out_shape = [out_shape, jax.ShapeDtypeStruct((N, N), jnp.bfloat16)]

    return pl.pallas_call(
        body,
        grid=(GRID,),
        in_specs=in_specs,
        out_specs=out_specs,
        out_shape=out_shape,
        scratch_shapes=[pltpu.VMEM((N, c_in_total), jnp.bfloat16)],
    )(A, *arrays, *scales, *weights)


_NV = N // 16


def _rank_masks_sc(scores, valid, k):
    use_valid = valid is not None
    mesh = plsc.VectorSubcoreMesh(core_axis_name="c", subcore_axis_name="s")
    n_in = 2 if use_valid else 1

    @functools.partial(
        pl.kernel, mesh=mesh,
        out_type=[jax.ShapeDtypeStruct((N,), jnp.float32),
                  jax.ShapeDtypeStruct((N,), jnp.float32)],
        scratch_types=[pltpu.VMEM((N + 32,), jnp.float32),
                       pltpu.VMEM((N,), jnp.float32),
                       pltpu.VMEM((128,), jnp.float32),
                       pltpu.VMEM((128,), jnp.float32)],
    )
    def rank_kernel(*refs):
        s_hbm = refs[0]
        v_hbm = refs[1] if use_valid else None
        m_hbm, sel_hbm = refs[n_in], refs[n_in + 1]
        sm_v, v_v, m_loc, sel_loc = refs[n_in + 2:n_in + 6]

        wid = lax.axis_index("s") * 2 + lax.axis_index("c")
        base = wid * 128
        neg = jnp.full((16,), -1.0, jnp.float32)
        pltpu.sync_copy(s_hbm, sm_v.at[pl.ds(16, N)])
        sm_v[pl.ds(0, 16)] = neg
        sm_v[pl.ds(N + 16, 16)] = neg
        if use_valid:
            pltpu.sync_copy(v_hbm, v_v)
            for q in range(_NV):
                sl = pl.ds(16 + q * 16, 16)
                sm_v[sl] = jnp.where(v_v[pl.ds(q * 16, 16)] > 0.0,
                                     sm_v[sl], -1.0)

        mines = [sm_v[pl.ds(16 + base + e * 16, 16)] for e in range(8)]

        NB = 4

        def jbody(j, carry):
            banks = [list(b) for b in carry]
            jb = j * 64
            for r0 in range(0, 64, NB):
                ws = [sm_v[pl.ds(16 + jb + r0 + q, 16)] for q in range(NB)]
                for e in range(8):
                    for q in range(NB):
                        banks[q][e] = jnp.where(ws[q] > mines[e],
                                                banks[q][e] + 1.0,
                                                banks[q][e])
            return tuple(tuple(b) for b in banks)

        zero = jnp.zeros((16,), jnp.float32)
        init = tuple((zero,) * 8 for _ in range(NB))
        banks = lax.fori_loop(0, _NV // 4, jbody, init)
        accs = [banks[0][e] + banks[1][e] + banks[2][e] + banks[3][e]
                for e in range(8)]

        for d in range(1, 16):
            w = sm_v[pl.ds(16 - d, 16)]
            for e in range(8):
                accs[e] = accs[e] + jnp.where(w > mines[e], 1.0, 0.0)

        kf = jnp.float32(k)
        for e in range(8):
            sel_e = jnp.where(accs[e] < kf, 1.0, 0.0)
            sl = pl.ds(e * 16, 16)
            sel_loc[sl] = sel_e
            m_loc[sl] = sel_e * mines[e]
        pltpu.sync_copy(m_loc, m_hbm.at[pl.ds(base, 128)])
        pltpu.sync_copy(sel_loc, sel_hbm.at[pl.ds(base, 128)])

    args = (scores, valid) if use_valid else (scores,)
    m, sel = rank_kernel(*args)
    return m.reshape(N, 1), sel.reshape(N, 1)


def kernel(A, x, params: dict[str, Any]):
    p = params

    def wt(lin):
        return lin["W"].T

    def bias(lin):
        return lin["b"].reshape(1, -1)

    x_s1, Ab = _fused_pass(
        A, [x], [],
        [wt(p["start_gcn_s1"]), bias(p["start_gcn_s1"])],
        parts=[[(0, 1.0, None, (0, None, None, None))]],
        outspecs=[dict(terms=[(0, None)], bias_id=1, act=None,
                       oscale_id=None, col=0, wout=DIM)],
        score=None, c_out=DIM, emit_bf16=True)

    is1 = p["is1"]
    w2 = [wt(p["s1_l1"]), bias(p["s1_l1"]),
          wt(is1["gcn1"]), bias(is1["gcn1"]),
          is1["disc"]["W"][0], is1["disc"]["b"].reshape(1, 1),
          wt(is1["fc"]), (is1["fc"]["b"] + is1["fc"]["bias2"]).reshape(1, -1),
          is1["fc"]["a"].reshape(1, 1)]
    pass2 = _fused_pass(
        Ab, [x_s1], [], w2,
        parts=[[(0, 1.0, None, (6, 7, "prelu", 8))], [(0, 1.0, None, None)]],
        outspecs=[dict(terms=[(1, 0)], bias_id=1, act="relu",
                       oscale_id=None, col=0, wout=DIM)],
        score=dict(hn_group=0, wg_id=2, bg_id=3, wd_id=4, bd_id=5,
                   h_group=0, col=DIM, hn_col=None),
        c_out=DIM + 1)
    x_s1a = pass2[:, 0:DIM]
    scores1 = pass2[:, DIM]
    m1, sel1 = _rank_masks_sc(scores1, None, K1)

    x_s2 = _fused_pass(
        Ab, [x_s1], [m1], [wt(p["start_gcn_s2"]), bias(p["start_gcn_s2"])],
        parts=[[(0, 1.0, 0, None)]],
        outspecs=[dict(terms=[(0, 0)], bias_id=1, act=None,
                       oscale_id=None, col=0, wout=DIM)],
        score=None, c_out=DIM)

    is2 = p["is2"]
    w4 = [wt(p["s2_l1"]), bias(p["s2_l1"]),
          wt(is2["gcn1"]), bias(is2["gcn1"]),
          is2["disc"]["W"][0], is2["disc"]["b"].reshape(1, 1),
          wt(is2["fc"]), (is2["fc"]["b"] + is2["fc"]["bias2"]).reshape(1, -1),
          is2["fc"]["a"].reshape(1, 1)]
    pass4 = _fused_pass(
        Ab, [x_s2], [sel1], w4,
        parts=[[(0, 1.0, 0, (6, 7, "prelu", 8))], [(0, 1.0, 0, None)]],
        outspecs=[dict(terms=[(1, 0)], bias_id=1, act="relu",
                       oscale_id=None, col=DIM, wout=DIM)],
        score=dict(hn_group=0, wg_id=2, bg_id=3, wd_id=4, bd_id=5,
                   h_group=0, col=2 * DIM, hn_col=0),
        c_out=2 * DIM + 1)
    xdown2 = pass4[:, 0:DIM]
    x_s2a = pass4[:, DIM:2 * DIM]
    scores2 = pass4[:, 2 * DIM]
    m2, sel2 = _rank_masks_sc(scores2, sel1.reshape(N), K2)

    x_s3a = _fused_pass(
        Ab, [x_s2], [m2], [wt(p["s3_l1"]), bias(p["s3_l1"])],
        parts=[[(0, 1.0, 0, None)]],
        outspecs=[dict(terms=[(0, 0)], bias_id=1, act="relu",
                       oscale_id=None, col=0, wout=DIM)],
        score=None, c_out=DIM)

    def cross_pass(xs1, xs2, xs3, wp12, wu21, wp23, wu32):
        w = [wt(wp12), bias(wp12), wt(wu21), bias(wu21),
             wt(wp23), bias(wp23), wt(wu32), bias(wu32)]
        out = _fused_pass(
            Ab, [xs1, xs2, xs3], [sel1, sel2, m1, m2], w,
            parts=[[(0, 1.0, None, None)], [(1, 1.0, 0, None)],
                   [(2, 1.0, 1, None)]],
            outspecs=[
                dict(terms=[(0, 0)], bias_id=1, act=None, oscale_id=2,
                     col=0, wout=DIM),
                dict(terms=[(1, 2)], bias_id=3, act=None, oscale_id=None,
                     col=DIM, wout=DIM),
                dict(terms=[(1, 4)], bias_id=5, act=None, oscale_id=3,
                     col=2 * DIM, wout=DIM),
                dict(terms=[(2, 6)], bias_id=7, act=None, oscale_id=None,
                     col=3 * DIM, wout=DIM),
            ],
            score=None, c_out=4 * DIM)
        return (out[:, 0:DIM], out[:, DIM:2 * DIM],
                out[:, 2 * DIM:3 * DIM], out[:, 3 * DIM:4 * DIM])

    x12, x21, x23, x32 = cross_pass(
        x_s1a, x_s2a, x_s3a,
        p["pool_s12_1"], p["unpool_s21_1"], p["pool_s23_1"], p["unpool_s32_1"])

    def tri_pass(arrays, scales, groups, l1, l2, l3):
        w = [wt(l1), bias(l1), wt(l2), bias(l2), wt(l3), bias(l3)]
        out = _fused_pass(
            Ab, arrays, scales, w, parts=groups,
            outspecs=[
                dict(terms=[(0, 0)], bias_id=1, act="relu", oscale_id=None,
                     col=0, wout=DIM),
                dict(terms=[(1, 2)], bias_id=3, act="relu", oscale_id=None,
                     col=DIM, wout=DIM),
                dict(terms=[(2, 4)], bias_id=5, act="relu", oscale_id=None,
                     col=2 * DIM, wout=DIM),
            ],
            score=None, c_out=3 * DIM)
        return out[:, 0:DIM], out[:, DIM:2 * DIM], out[:, 2 * DIM:3 * DIM]

    x_s1b, x_s2b, x_s3b = tri_pass(
        [x_s1a, x21, x_s1, x_s2a, x12, x32, x_s2, x_s3a, x23],
        [sel1, sel2, m2],
        [
            [(0, 1.0, None, None), (1, 1.0, None, None), (2, 1.0, None, None)],
            [(3, 1.0, 0, None), (4, 0.5, None, None), (5, 0.5, 0, None),
             (6, 1.0, 0, None)],
            [(7, 1.0, 1, None), (8, 1.0, None, None), (6, 1.0, 2, None)],
        ],
        p["s1_l2"], p["s2_l2"], p["s3_l2"])

    x12b, x21b, x23b, x32b = cross_pass(
        x_s1b, x_s2b, x_s3b,
        p["pool_s12_2"], p["unpool_s21_2"], p["pool_s23_2"], p["unpool_s32_2"])

    x_s1f, x_s2f, x_s3f = tri_pass(
        [x_s1b, x21b, x_s2b, x12b, x32b, x_s3b, x23b],
        [sel1, sel2],
        [
            [(0, 1.0, None, None), (1, 0.05, None, None)],
            [(2, 1.0, 0, None), (3, 0.025, None, None), (4, 0.025, 0, None)],
            [(5, 1.0, 1, None), (6, 0.05, None, None)],
        ],
        p["s1_l3"], p["s2_l3"], p["s3_l3"])

    u32e = _fused_pass(
        Ab, [x_s3f], [sel2],
        [wt(p["unpool_s32_end"]), bias(p["unpool_s32_end"])],
        parts=[[(0, 1.0, 0, None)]],
        outspecs=[dict(terms=[(0, 0)], bias_id=1, act=None,
                       oscale_id=None, col=0, wout=DIM)],
        score=None, c_out=DIM)

    x_s2out = _fused_pass(
        Ab, [x_s2f, u32e, xdown2], [sel1],
        [wt(p["unpool_s21_end"]), bias(p["unpool_s21_end"])],
        parts=[[(0, 1.0, 0, None), (1, 1.0, 0, None), (2, 1.0, 0, None)]],
        outspecs=[dict(terms=[(0, 0)], bias_id=1, act=None,
                       oscale_id=None, col=0, wout=DIM)],
        score=None, c_out=DIM)

    wend = wt(p["end_gcn"])
    out = _fused_pass(
        Ab, [x_s1f, x_s2out], [],
        [wend[0:DIM, :], wend[DIM:2 * DIM, :], bias(p["end_gcn"])],
        parts=[[(0, 1.0, None, None)], [(1, 1.0, None, None)]],
        outspecs=[dict(terms=[(0, 0), (1, 1)], bias_id=2, act=None,
                       oscale_id=None, col=0, wout=256)],
        score=None, c_out=256)
    return out


# speedup vs baseline: 1.4405x; 1.0107x over previous
import functools
from typing import Any

import jax
import jax.numpy as jnp
from jax import lax
from jax.experimental import pallas as pl
from jax.experimental.pallas import tpu as pltpu
from jax.experimental.pallas import tpu_sc as plsc

N = 4096
DIM = 48
K1 = int(0.8 * N)
K2 = int(0.7 * K1)
BM = 1024
GRID = N // BM


def _fused_pass(A, arrays, scales, weights, parts, outspecs, score, c_out,
                emit_bf16=False):
    n_arr = len(arrays)
    n_sc = len(scales)

    def _entry_w(e):
        return (arrays[e[0]].shape[1] if e[3] is None
                else weights[e[3][0]].shape[1])

    group_w = [_entry_w(g[0]) for g in parts]
    c_in_total = sum(group_w)
    group_off = [sum(group_w[:gi]) for gi in range(len(parts))]

    def body(*refs):
        a_ref = refs[0]
        arr_refs = refs[1:1 + n_arr]
        sc_refs = refs[1 + n_arr:1 + n_arr + n_sc]
        w_refs = refs[1 + n_arr + n_sc:1 + n_arr + n_sc + len(weights)]
        x_scr = refs[-1]
        if emit_bf16:
            out_ref, ab_ref = refs[-3], refs[-2]
        else:
            out_ref = refs[-2]
        i = pl.program_id(0)

        def load_entry(entry, row_slice=None):
            aid, coef, sid, pre = entry
            v = (arr_refs[aid][...] if row_slice is None
                 else arr_refs[aid][row_slice, :])
            if pre is not None:
                wid, bid, act, a_id = pre
                v = jnp.dot(v, w_refs[wid][...],
                            preferred_element_type=jnp.float32)
                if bid is not None:
                    v = v + w_refs[bid][...]
                if act == "prelu":
                    a = w_refs[a_id][0, 0]
                    v = jnp.where(v >= 0.0, v, a * v)
            if sid is not None:
                s = (sc_refs[sid][...] if row_slice is None
                     else sc_refs[sid][row_slice, :])
                v = v * s
            if coef != 1.0:
                v = v * coef
            return v

        @pl.when(i == 0)
        def _assemble():
            for gi, group in enumerate(parts):
                acc = None
                for entry in group:
                    v = load_entry(entry)
                    acc = v if acc is None else acc + v
                x_scr[:, group_off[gi]:group_off[gi] + group_w[gi]] = (
                    acc.astype(x_scr.dtype))

        a_blk = a_ref[...]
        if emit_bf16:
            a_blk = a_blk.astype(jnp.bfloat16)
            ab_ref[...] = a_blk
        raws = []
        for gi in range(len(parts)):
            gv = x_scr[:, group_off[gi]:group_off[gi] + group_w[gi]]
            raws.append(jnp.dot(a_blk, gv,
                                preferred_element_type=jnp.float32))

        for spec in outspecs:
            y = None
            for (gi, wid) in spec["terms"]:
                t = raws[gi] if wid is None else jnp.dot(
                    raws[gi], w_refs[wid][...],
                    preferred_element_type=jnp.float32)
                y = t if y is None else y + t
            y = y + w_refs[spec["bias_id"]][...]
            if spec["act"] == "relu":
                y = jnp.maximum(y, 0.0)
            if spec["oscale_id"] is not None:
                y = y * sc_refs[spec["oscale_id"]][pl.ds(i * BM, BM), :]
            out_ref[:, spec["col"]:spec["col"] + spec["wout"]] = y

        if score is not None:
            hn = jnp.dot(raws[score["hn_group"]], w_refs[score["wg_id"]][...],
                         preferred_element_type=jnp.float32)
            hn = hn + w_refs[score["bg_id"]][...]
            if score["hn_col"] is not None:
                out_ref[:, score["hn_col"]:score["hn_col"] + DIM] = hn
            xs = jax.nn.sigmoid(hn)
            h = None
            for entry in parts[score["h_group"]]:
                v = load_entry(entry, row_slice=pl.ds(i * BM, BM))
                h = v if h is None else h + v
            hw = jnp.dot(h, w_refs[score["wd_id"]][...],
                         preferred_element_type=jnp.float32)
            t = jnp.sum(hw * xs, axis=1, keepdims=True)
            t = t + w_refs[score["bd_id"]][...]
            out_ref[:, score["col"]:score["col"] + 1] = jax.nn.sigmoid(t)

    in_specs = [pl.BlockSpec((BM, N), lambda i: (i, 0))]
    for a in arrays:
        w = a.shape[1]
        in_specs.append(pl.BlockSpec((N, w), lambda i: (0, 0)))
    for _ in scales:
        in_specs.append(pl.BlockSpec((N, 1), lambda i: (0, 0)))
    for wgt in weights:
        in_specs.append(pl.BlockSpec(wgt.shape, lambda i: (0, 0)))

    out_specs = pl.BlockSpec((BM, c_out), lambda i: (i, 0))
    out_shape = jax.ShapeDtypeStruct((N, c_out), jnp.float32)
    if emit_bf16:
        out_specs = [out_specs, pl.BlockSpec((BM, N), lambda i: (i, 0))]
        out_shape = [out_shape, jax.ShapeDtypeStruct((N, N), jnp.bfloat16)]

    return pl.pallas_call(
        body,
        grid=(GRID,),
        in_specs=in_specs,
        out_specs=out_specs,
        out_shape=out_shape,
        scratch_shapes=[pltpu.VMEM((N, c_in_total), jnp.bfloat16)],
    )(A, *arrays, *scales, *weights)


_NV = N // 16


def _rank_masks_sc(scores, valid, k):
    use_valid = valid is not None
    mesh = plsc.VectorSubcoreMesh(core_axis_name="c", subcore_axis_name="s")
    n_in = 2 if use_valid else 1

    @functools.partial(
        pl.kernel, mesh=mesh,
        out_type=[jax.ShapeDtypeStruct((N,), jnp.float32),
                  jax.ShapeDtypeStruct((N,), jnp.float32)],
        scratch_types=[pltpu.VMEM((N + 32,), jnp.float32),
                       pltpu.VMEM((N,), jnp.float32),
                       pltpu.VMEM((128,), jnp.float32),
                       pltpu.VMEM((128,), jnp.float32)],
    )
    def rank_kernel(*refs):
        s_hbm = refs[0]
        v_hbm = refs[1] if use_valid else None
        m_hbm, sel_hbm = refs[n_in], refs[n_in + 1]
        sm_v, v_v, m_loc, sel_loc = refs[n_in + 2:n_in + 6]

        wid = lax.axis_index("s") * 2 + lax.axis_index("c")
        base = wid * 128
        neg = jnp.full((16,), -1.0, jnp.float32)
        pltpu.sync_copy(s_hbm, sm_v.at[pl.ds(16, N)])
        sm_v[pl.ds(0, 16)] = neg
        sm_v[pl.ds(N + 16, 16)] = neg
        if use_valid:
            pltpu.sync_copy(v_hbm, v_v)
            for q in range(_NV):
                sl = pl.ds(16 + q * 16, 16)
                sm_v[sl] = jnp.where(v_v[pl.ds(q * 16, 16)] > 0.0,
                                     sm_v[sl], -1.0)

        mines = [sm_v[pl.ds(16 + base + e * 16, 16)] for e in range(8)]

        NB = 4

        def jbody(j, carry):
            banks = [list(b) for b in carry]
            jb = j * 64
            for r0 in range(0, 64, NB):
                ws = [sm_v[pl.ds(16 + jb + r0 + q, 16)] for q in range(NB)]
                for e in range(8):
                    for q in range(NB):
                        banks[q][e] = jnp.where(ws[q] > mines[e],
                                                banks[q][e] + 1.0,
                                                banks[q][e])
            return tuple(tuple(b) for b in banks)

        zero = jnp.zeros((16,), jnp.float32)
        init = tuple((zero,) * 8 for _ in range(NB))
        banks = lax.fori_loop(0, _NV // 4, jbody, init)
        accs = [banks[0][e] + banks[1][e] + banks[2][e] + banks[3][e]
                for e in range(8)]

        for d in range(1, 16):
            w = sm_v[pl.ds(16 - d, 16)]
            for e in range(8):
                accs[e] = accs[e] + jnp.where(w > mines[e], 1.0, 0.0)

        kf = jnp.float32(k)
        for e in range(8):
            sel_e = jnp.where(accs[e] < kf, 1.0, 0.0)
            sl = pl.ds(e * 16, 16)
            sel_loc[sl] = sel_e
            m_loc[sl] = sel_e * mines[e]
        pltpu.sync_copy(m_loc, m_hbm.at[pl.ds(base, 128)])
        pltpu.sync_copy(sel_loc, sel_hbm.at[pl.ds(base, 128)])

    args = (scores, valid) if use_valid else (scores,)
    m, sel = rank_kernel(*args)
    return m.reshape(N, 1), sel.reshape(N, 1)


def kernel(A, x, params: dict[str, Any]):
    p = params

    def wt(lin):
        return lin["W"].T

    def bias(lin):
        return lin["b"].reshape(1, -1)

    x_s1, Ab = _fused_pass(
        A, [x], [],
        [wt(p["start_gcn_s1"]), bias(p["start_gcn_s1"])],
        parts=[[(0, 1.0, None, (0, None, None, None))]],
        outspecs=[dict(terms=[(0, None)], bias_id=1, act=None,
                       oscale_id=None, col=0, wout=DIM)],
        score=None, c_out=DIM, emit_bf16=True)

    is1 = p["is1"]
    w2 = [wt(p["s1_l1"]), bias(p["s1_l1"]),
          wt(is1["gcn1"]), bias(is1["gcn1"]),
          is1["disc"]["W"][0], is1["disc"]["b"].reshape(1, 1),
          wt(is1["fc"]), (is1["fc"]["b"] + is1["fc"]["bias2"]).reshape(1, -1),
          is1["fc"]["a"].reshape(1, 1)]
    pass2 = _fused_pass(
        Ab, [x_s1], [], w2,
        parts=[[(0, 1.0, None, (6, 7, "prelu", 8))], [(0, 1.0, None, None)]],
        outspecs=[dict(terms=[(1, 0)], bias_id=1, act="relu",
                       oscale_id=None, col=0, wout=DIM)],
        score=dict(hn_group=0, wg_id=2, bg_id=3, wd_id=4, bd_id=5,
                   h_group=0, col=DIM, hn_col=None),
        c_out=DIM + 1)
    x_s1a = pass2[:, 0:DIM]
    scores1 = pass2[:, DIM]
    m1, sel1 = _rank_masks_sc(scores1, None, K1)

    x_s2 = _fused_pass(
        Ab, [x_s1], [m1], [wt(p["start_gcn_s2"]), bias(p["start_gcn_s2"])],
        parts=[[(0, 1.0, 0, None)]],
        outspecs=[dict(terms=[(0, 0)], bias_id=1, act=None,
                       oscale_id=None, col=0, wout=DIM)],
        score=None, c_out=DIM)

    is2 = p["is2"]
    w4 = [wt(p["s2_l1"]), bias(p["s2_l1"]),
          wt(is2["gcn1"]), bias(is2["gcn1"]),
          is2["disc"]["W"][0], is2["disc"]["b"].reshape(1, 1),
          wt(is2["fc"]), (is2["fc"]["b"] + is2["fc"]["bias2"]).reshape(1, -1),
          is2["fc"]["a"].reshape(1, 1)]
    pass4 = _fused_pass(
        Ab, [x_s2], [sel1], w4,
        parts=[[(0, 1.0, 0, (6, 7, "prelu", 8))], [(0, 1.0, 0, None)]],
        outspecs=[dict(terms=[(1, 0)], bias_id=1, act="relu",
                       oscale_id=None, col=DIM, wout=DIM)],
        score=dict(hn_group=0, wg_id=2, bg_id=3, wd_id=4, bd_id=5,
                   h_group=0, col=2 * DIM, hn_col=0),
        c_out=2 * DIM + 1)
    xdown2 = pass4[:, 0:DIM]
    x_s2a = pass4[:, DIM:2 * DIM]
    scores2 = pass4[:, 2 * DIM]
    m2, sel2 = _rank_masks_sc(scores2, sel1.reshape(N), K2)

    x_s3a = _fused_pass(
        Ab, [x_s2], [m2], [wt(p["s3_l1"]), bias(p["s3_l1"])],
        parts=[[(0, 1.0, 0, None)]],
        outspecs=[dict(terms=[(0, 0)], bias_id=1, act="relu",
                       oscale_id=None, col=0, wout=DIM)],
        score=None, c_out=DIM)

    def cross_pass(xs1, xs2, xs3, wp12, wu21, wp23, wu32):
        w = [wt(wp12), bias(wp12), wt(wu21), bias(wu21),
             wt(wp23), bias(wp23), wt(wu32), bias(wu32)]
        out = _fused_pass(
            Ab, [xs1, xs2, xs3], [sel1, sel2, m1, m2], w,
            parts=[[(0, 1.0, None, None)], [(1, 1.0, 0, None)],
                   [(2, 1.0, 1, None)]],
            outspecs=[
                dict(terms=[(0, 0)], bias_id=1, act=None, oscale_id=2,
                     col=0, wout=DIM),
                dict(terms=[(1, 2)], bias_id=3, act=None, oscale_id=None,
                     col=DIM, wout=DIM),
                dict(terms=[(1, 4)], bias_id=5, act=None, oscale_id=3,
                     col=2 * DIM, wout=DIM),
                dict(terms=[(2, 6)], bias_id=7, act=None, oscale_id=None,
                     col=3 * DIM, wout=DIM),
            ],
            score=None, c_out=4 * DIM)
        return (out[:, 0:DIM], out[:, DIM:2 * DIM],
                out[:, 2 * DIM:3 * DIM], out[:, 3 * DIM:4 * DIM])

    x12, x21, x23, x32 = cross_pass(
        x_s1a, x_s2a, x_s3a,
        p["pool_s12_1"], p["unpool_s21_1"], p["pool_s23_1"], p["unpool_s32_1"])

    def tri_pass(arrays, scales, groups, l1, l2, l3):
        w = [wt(l1), bias(l1), wt(l2), bias(l2), wt(l3), bias(l3)]
        out = _fused_pass(
            Ab, arrays, scales, w, parts=groups,
            outspecs=[
                dict(terms=[(0, 0)], bias_id=1, act="relu", oscale_id=None,
                     col=0, wout=DIM),
                dict(terms=[(1, 2)], bias_id=3, act="relu", oscale_id=None,
                     col=DIM, wout=DIM),
                dict(terms=[(2, 4)], bias_id=5, act="relu", oscale_id=None,
                     col=2 * DIM, wout=DIM),
            ],
            score=None, c_out=3 * DIM)
        return out[:, 0:DIM], out[:, DIM:2 * DIM], out[:, 2 * DIM:3 * DIM]

    x_s1b, x_s2b, x_s3b = tri_pass(
        [x_s1a, x21, x_s1, x_s2a, x12, x32, x_s2, x_s3a, x23],
        [sel1, sel2, m2],
        [
            [(0, 1.0, None, None), (1, 1.0, None, None), (2, 1.0, None, None)],
            [(3, 1.0, 0, None), (4, 0.5, None, None), (5, 0.5, 0, None),
             (6, 1.0, 0, None)],
            [(7, 1.0, 1, None), (8, 1.0, None, None), (6, 1.0, 2, None)],
        ],
        p["s1_l2"], p["s2_l2"], p["s3_l2"])

    x12b, x21b, x23b, x32b = cross_pass(
        x_s1b, x_s2b, x_s3b,
        p["pool_s12_2"], p["unpool_s21_2"], p["pool_s23_2"], p["unpool_s32_2"])

    x_s1f, x_s2f, x_s3f = tri_pass(
        [x_s1b, x21b, x_s2b, x12b, x32b, x_s3b, x23b],
        [sel1, sel2],
        [
            [(0, 1.0, None, None), (1, 0.05, None, None)],
            [(2, 1.0, 0, None), (3, 0.025, None, None), (4, 0.025, 0, None)],
            [(5, 1.0, 1, None), (6, 0.05, None, None)],
        ],
        p["s1_l3"], p["s2_l3"], p["s3_l3"])

    u32e = _fused_pass(
        Ab, [x_s3f], [sel2],
        [wt(p["unpool_s32_end"]), bias(p["unpool_s32_end"])],
        parts=[[(0, 1.0, 0, None)]],
        outspecs=[dict(terms=[(0, 0)], bias_id=1, act=None,
                       oscale_id=None, col=0, wout=DIM)],
        score=None, c_out=DIM)

    x_s2out = _fused_pass(
        Ab, [x_s2f, u32e, xdown2], [sel1],
        [wt(p["unpool_s21_end"]), bias(p["unpool_s21_end"])],
        parts=[[(0, 1.0, 0, None), (1, 1.0, 0, None), (2, 1.0, 0, None)]],
        outspecs=[dict(terms=[(0, 0)], bias_id=1, act=None,
                       oscale_id=None, col=0, wout=DIM)],
        score=None, c_out=DIM)

    wend = wt(p["end_gcn"])
    out = _fused_pass(
        Ab, [x_s1f, x_s2out], [],
        [wend[0:DIM, :], wend[DIM:2 * DIM, :], bias(p["end_gcn"])],
        parts=[[(0, 1.0, None, None)], [(1, 1.0, None, None)]],
        outspecs=[dict(terms=[(0, 0), (1, 1)], bias_id=2, act=None,
                       oscale_id=None, col=0, wout=256)],
        score=None, c_out=256)
    return out
